# baseline (speedup 1.0000x reference)
import functools

import jax
import jax.numpy as jnp
from jax import lax
from jax.experimental import pallas as pl
from jax.experimental.pallas import tpu as pltpu
from jax.experimental.pallas import tpu_sc as plsc

F32 = jnp.float32


def _sc_info():
    try:
        info = plsc.get_sparse_core_info()
        return info.num_cores, info.num_subcores
    except Exception:
        return 2, 16


def _build_gather_sum(P, n_rows, n_chunks, C, NC, NS):
    mesh = plsc.VectorSubcoreMesh(core_axis_name="c", subcore_axis_name="s")
    half = n_chunks // 2

    def body(msgs_hbm, i0_hbm, i1_hbm, i2_hbm, out_hbm,
             i0_v, i1_v, i2_v, rows_v, gsem0, gsem1, wsem0, wsem1):
        wid = lax.axis_index("s") * NC + lax.axis_index("c")
        base0 = wid * n_rows
        pltpu.sync_copy(i0_hbm.at[pl.ds(base0, n_rows)], i0_v)
        pltpu.sync_copy(i1_hbm.at[pl.ds(base0, n_rows)], i1_v)
        pltpu.sync_copy(i2_hbm.at[pl.ds(base0, n_rows)], i2_v)
        idx_vs = (i0_v, i1_v, i2_v)
        gsems = (gsem0, gsem1)
        wsems = (wsem0, wsem1)

        def fire(koff, s):
            for j in range(3):
                pltpu.async_copy(
                    msgs_hbm.at[idx_vs[j].at[pl.ds(koff * C, C)]],
                    rows_v.at[s, j], gsems[s])

        def wait_gathers(koff, s):
            for j in range(3):
                pltpu.make_async_copy(
                    msgs_hbm.at[idx_vs[j].at[pl.ds(koff * C, C)]],
                    rows_v.at[s, j], gsems[s]).wait()

        def sum_wb(koff, s):
            def row(r, c2):
                for l in range(8):
                    sl = pl.ds(l * 16, 16)
                    rows_v[s, 0, r, sl] = (rows_v[s, 0, r, sl]
                                           + rows_v[s, 1, r, sl]
                                           + rows_v[s, 2, r, sl])
                return c2

            lax.fori_loop(0, C, row, 0)
            pltpu.async_copy(rows_v.at[s, 0],
                             out_hbm.at[pl.ds(base0 + koff * C, C)], wsems[s])

        def wait_wb(koff, s):
            pltpu.make_async_copy(rows_v.at[s, 0],
                                  out_hbm.at[pl.ds(base0 + koff * C, C)],
                                  wsems[s]).wait()

        fire(0, 0)

        def pair(k2, carry):
            c0 = 2 * k2

            @pl.when(k2 >= 1)
            def _():
                wait_wb(c0 - 1, 1)

            fire(c0 + 1, 1)
            wait_gathers(c0, 0)
            sum_wb(c0, 0)

            @pl.when(k2 + 1 < half)
            def _():
                wait_wb(c0, 0)
                fire(c0 + 2, 0)

            wait_gathers(c0 + 1, 1)
            sum_wb(c0 + 1, 1)
            return carry

        lax.fori_loop(0, half, pair, 0)
        wait_wb(2 * half - 2, 0)
        wait_wb(2 * half - 1, 1)

    return pl.kernel(
        body,
        out_type=jax.ShapeDtypeStruct((P, 128), F32),
        mesh=mesh,
        scratch_types=[
            pltpu.VMEM((n_rows,), jnp.int32),
            pltpu.VMEM((n_rows,), jnp.int32),
            pltpu.VMEM((n_rows,), jnp.int32),
            pltpu.VMEM((2, 3, C, 128), F32),
            pltpu.SemaphoreType.DMA,
            pltpu.SemaphoreType.DMA,
            pltpu.SemaphoreType.DMA,
            pltpu.SemaphoreType.DMA,
        ],
    )


def _local_potentials(f_bond_p, WlT, BM):
    EP, K = f_bond_p.shape
    H = WlT.shape[1]

    def body(fb_ref, w_ref, lp_ref, msg_ref):
        lp = jnp.dot(fb_ref[...], w_ref[...], preferred_element_type=F32)
        lp_ref[...] = lp
        msg_ref[...] = jnp.maximum(lp, 0.0)

    return pl.pallas_call(
        body,
        grid=(EP // BM,),
        in_specs=[
            pl.BlockSpec((BM, K), lambda i: (i, 0)),
            pl.BlockSpec((K, H), lambda i: (0, 0)),
        ],
        out_specs=[
            pl.BlockSpec((BM, H), lambda i: (i, 0)),
            pl.BlockSpec((BM, H), lambda i: (i, 0)),
        ],
        out_shape=[
            jax.ShapeDtypeStruct((EP, H), F32),
            jax.ShapeDtypeStruct((EP, H), F32),
        ],
    )(f_bond_p, WlT)


def _gru_update(sn, lp, msg, WmT, WihT, WhhT, bih, bhh, BM):
    EP, H = sn.shape

    def body(sn_ref, lp_ref, msg_ref, wm_ref, wi_ref, wh_ref, bi_ref, bh_ref, out_ref):
        nb = jnp.dot(sn_ref[...], wm_ref[...], preferred_element_type=F32)
        new = jnp.maximum(lp_ref[...] + nb, 0.0)
        h = msg_ref[...]
        gi = jnp.dot(new, wi_ref[...], preferred_element_type=F32) + bi_ref[...]
        gh = jnp.dot(h, wh_ref[...], preferred_element_type=F32) + bh_ref[...]
        r = jax.nn.sigmoid(gi[:, :H] + gh[:, :H])
        z = jax.nn.sigmoid(gi[:, H:2 * H] + gh[:, H:2 * H])
        n = jnp.tanh(gi[:, 2 * H:] + r * gh[:, 2 * H:])
        out_ref[...] = (1.0 - z) * n + z * h

        @pl.when(pl.program_id(0) == 0)
        def _():
            out_ref[0:1, :] = jnp.zeros((1, H), F32)

    return pl.pallas_call(
        body,
        grid=(EP // BM,),
        in_specs=[
            pl.BlockSpec((BM, H), lambda i: (i, 0)),
            pl.BlockSpec((BM, H), lambda i: (i, 0)),
            pl.BlockSpec((BM, H), lambda i: (i, 0)),
            pl.BlockSpec((H, H), lambda i: (0, 0)),
            pl.BlockSpec((H, 3 * H), lambda i: (0, 0)),
            pl.BlockSpec((H, 3 * H), lambda i: (0, 0)),
            pl.BlockSpec((1, 3 * H), lambda i: (0, 0)),
            pl.BlockSpec((1, 3 * H), lambda i: (0, 0)),
        ],
        out_specs=pl.BlockSpec((BM, H), lambda i: (i, 0)),
        out_shape=jax.ShapeDtypeStruct((EP, H), F32),
    )(sn, lp, msg, WmT, WihT, WhhT, bih, bhh)


def _node_embedding(f_nuc_p, nnm, W1T, W2T, BM):
    NP, K = f_nuc_p.shape
    H = W2T.shape[1]

    def body(fn_ref, nm_ref, w1_ref, w2_ref, out_ref):
        acc = jnp.dot(fn_ref[...], w1_ref[...], preferred_element_type=F32)
        acc = acc + jnp.dot(nm_ref[...], w2_ref[...], preferred_element_type=F32)
        out_ref[...] = jnp.maximum(acc, 0.0)

    return pl.pallas_call(
        body,
        grid=(NP // BM,),
        in_specs=[
            pl.BlockSpec((BM, K), lambda i: (i, 0)),
            pl.BlockSpec((BM, H), lambda i: (i, 0)),
            pl.BlockSpec((K, H), lambda i: (0, 0)),
            pl.BlockSpec((H, H), lambda i: (0, 0)),
        ],
        out_specs=pl.BlockSpec((BM, H), lambda i: (i, 0)),
        out_shape=jax.ShapeDtypeStruct((NP, H), F32),
    )(f_nuc_p, nnm, W1T, W2T)


def _bilstm_maxpool(ne_t, WifT, WhfT, bf, WibT, WhbT, bb, T_b):
    Lq, Bq, H = ne_t.shape
    HH = WhfT.shape[0]
    G = Lq // T_b

    def body(nef_ref, neb_ref, wif_ref, whf_ref, bf_ref, wib_ref, whb_ref, bb_ref,
             out_ref, hf_s, cf_s, hb_s, cb_s, mf_s, mb_s):
        i = pl.program_id(0)

        @pl.when(i == 0)
        def _():
            z = jnp.zeros((Bq, HH), F32)
            hf_s[...] = z
            cf_s[...] = z
            hb_s[...] = z
            cb_s[...] = z
            m0 = jnp.full((Bq, HH), -jnp.inf, F32)
            mf_s[...] = m0
            mb_s[...] = m0

        def one_dir(x, h, c, wi_ref, wh_ref, b_ref):
            g = (jnp.dot(x, wi_ref[...], preferred_element_type=F32)
                 + jnp.dot(h, wh_ref[...], preferred_element_type=F32)
                 + b_ref[...])
            ig = jax.nn.sigmoid(g[:, :HH])
            fg = jax.nn.sigmoid(g[:, HH:2 * HH])
            gg = jnp.tanh(g[:, 2 * HH:3 * HH])
            og = jax.nn.sigmoid(g[:, 3 * HH:])
            c = fg * c + ig * gg
            h = og * jnp.tanh(c)
            return h, c

        def step(tt, carry):
            hf, cf, hb, cb, mf, mb = carry
            hf, cf = one_dir(nef_ref[tt], hf, cf, wif_ref, whf_ref, bf_ref)
            mf = jnp.maximum(mf, hf)
            hb, cb = one_dir(neb_ref[T_b - 1 - tt], hb, cb, wib_ref, whb_ref, bb_ref)
            mb = jnp.maximum(mb, hb)
            return hf, cf, hb, cb, mf, mb

        init = (hf_s[...], cf_s[...], hb_s[...], cb_s[...], mf_s[...], mb_s[...])
        hf, cf, hb, cb, mf, mb = lax.fori_loop(0, T_b, step, init)
        hf_s[...] = hf
        cf_s[...] = cf
        hb_s[...] = hb
        cb_s[...] = cb
        mf_s[...] = mf
        mb_s[...] = mb

        @pl.when(i == G - 1)
        def _():
            out_ref[...] = jnp.concatenate([mf, mb], axis=1)

    return pl.pallas_call(
        body,
        grid=(G,),
        in_specs=[
            pl.BlockSpec((T_b, Bq, H), lambda i: (i, 0, 0)),
            pl.BlockSpec((T_b, Bq, H), lambda i: (G - 1 - i, 0, 0)),
            pl.BlockSpec((H, 4 * HH), lambda i: (0, 0)),
            pl.BlockSpec((HH, 4 * HH), lambda i: (0, 0)),
            pl.BlockSpec((1, 4 * HH), lambda i: (0, 0)),
            pl.BlockSpec((H, 4 * HH), lambda i: (0, 0)),
            pl.BlockSpec((HH, 4 * HH), lambda i: (0, 0)),
            pl.BlockSpec((1, 4 * HH), lambda i: (0, 0)),
        ],
        out_specs=pl.BlockSpec((Bq, 2 * HH), lambda i: (0, 0)),
        out_shape=jax.ShapeDtypeStruct((Bq, 2 * HH), F32),
        scratch_shapes=[pltpu.VMEM((Bq, HH), F32)] * 6,
    )(ne_t, ne_t, WifT, WhfT, bf, WibT, WhbT, bb)


def _pad_rows(x, P):
    n = x.shape[0]
    if n == P:
        return x
    return jnp.concatenate(
        [x, jnp.zeros((P - n,) + x.shape[1:], x.dtype)], axis=0)


def kernel(f_nuc, f_bond, node_graph, message_graph, all_bonds, scope,
           W_local, W_msg, W_node_emb,
           gru_w_ih, gru_w_hh, gru_b_ih, gru_b_hh,
           lstm_w_ih_f, lstm_w_hh_f, lstm_b_ih_f, lstm_b_hh_f,
           lstm_w_ih_b, lstm_w_hh_b, lstm_b_ih_b, lstm_b_hh_b):
    E = f_bond.shape[0]
    N = f_nuc.shape[0]
    H = W_msg.shape[0]
    B = scope.shape[0]
    L = N // B
    HH = lstm_w_hh_f.shape[1]
    NC, NS = _sc_info()
    NW = NC * NS
    C = 128
    unit = NW * C * 2

    EP = ((E + unit - 1) // unit) * unit
    NP = ((N + unit - 1) // unit) * unit
    e_rows = EP // NW
    n_rows = NP // NW

    f_bond_p = _pad_rows(f_bond.astype(F32), EP)
    f_nuc_p = _pad_rows(f_nuc.astype(F32), NP)
    mg = _pad_rows(message_graph.astype(jnp.int32), EP)
    ng = _pad_rows(node_graph.astype(jnp.int32), NP)
    i0, i1, i2 = mg[:, 0], mg[:, 1], mg[:, 2]
    n0, n1, n2 = ng[:, 0], ng[:, 1], ng[:, 2]

    WlT = W_local.T.astype(F32)
    WmT = W_msg.T
    WihT = gru_w_ih.T
    WhhT = gru_w_hh.T
    bih = gru_b_ih.reshape(1, 3 * H)
    bhh = gru_b_hh.reshape(1, 3 * H)
    W1T = W_node_emb[:, :4].T
    W2T = W_node_emb[:, 4:].T
    WifT = lstm_w_ih_f.T
    WhfT = lstm_w_hh_f.T
    bf = (lstm_b_ih_f + lstm_b_hh_f).reshape(1, 4 * HH)
    WibT = lstm_w_ih_b.T
    WhbT = lstm_w_hh_b.T
    bb = (lstm_b_ih_b + lstm_b_hh_b).reshape(1, 4 * HH)

    BM = 2048
    lp, msgs = _local_potentials(f_bond_p, WlT, BM)

    gsum_e = _build_gather_sum(EP, e_rows, e_rows // C, C, NC, NS)
    for _ in range(2):
        sn = gsum_e(msgs, i0, i1, i2)
        msgs = _gru_update(sn, lp, msgs, WmT, WihT, WhhT, bih, bhh, BM)

    gsum_n = _build_gather_sum(NP, n_rows, n_rows // C, C, NC, NS)
    nnm = gsum_n(msgs, n0, n1, n2)

    ne = _node_embedding(f_nuc_p, nnm, W1T, W2T, BM)
    ne_t = ne[:N].reshape(B, L, H).transpose(1, 0, 2)

    T_b = 1
    for d in range(min(25, L), 0, -1):
        if L % d == 0:
            T_b = d
            break
    rep = _bilstm_maxpool(ne_t, WifT, WhfT, bf, WibT, WhbT, bb, T_b)
    return rep


# speedup vs baseline: 1.0020x; 1.0020x over previous
import functools

import jax
import jax.numpy as jnp
from jax import lax
from jax.experimental import pallas as pl
from jax.experimental.pallas import tpu as pltpu
from jax.experimental.pallas import tpu_sc as plsc

F32 = jnp.float32


def _sc_info():
    try:
        info = plsc.get_sparse_core_info()
        return info.num_cores, info.num_subcores
    except Exception:
        return 2, 16


def _build_gather_sum(P, n_rows, n_chunks, C, NC, NS):
    mesh = plsc.VectorSubcoreMesh(core_axis_name="c", subcore_axis_name="s")
    half = n_chunks // 2

    def body(msgs_hbm, i0_hbm, i1_hbm, i2_hbm, out_hbm,
             ia0, ia1, ia2, ib0, ib1, ib2, rows_v, gsem0, gsem1, wsem0, wsem1):
        wid = lax.axis_index("s") * NC + lax.axis_index("c")
        base0 = wid * n_rows
        idx_hbms = (i0_hbm, i1_hbm, i2_hbm)
        idx_vs = ((ia0, ia1, ia2), (ib0, ib1, ib2))
        gsems = (gsem0, gsem1)
        wsems = (wsem0, wsem1)

        def stage(koff, s):
            base = base0 + koff * C
            for j in range(3):
                pltpu.sync_copy(idx_hbms[j].at[pl.ds(base, C)], idx_vs[s][j])

        def fire(s):
            for j in range(3):
                pltpu.async_copy(msgs_hbm.at[idx_vs[s][j]],
                                 rows_v.at[s, j], gsems[s])

        def wait_gathers(s):
            for j in range(3):
                pltpu.make_async_copy(msgs_hbm.at[idx_vs[s][j]],
                                      rows_v.at[s, j], gsems[s]).wait()

        def sum_wb(koff, s):
            def row(r, c2):
                for l in range(8):
                    sl = pl.ds(l * 16, 16)
                    rows_v[s, 0, r, sl] = (rows_v[s, 0, r, sl]
                                           + rows_v[s, 1, r, sl]
                                           + rows_v[s, 2, r, sl])
                return c2

            lax.fori_loop(0, C, row, 0)
            pltpu.async_copy(rows_v.at[s, 0],
                             out_hbm.at[pl.ds(base0 + koff * C, C)], wsems[s])

        def wait_wb(koff, s):
            pltpu.make_async_copy(rows_v.at[s, 0],
                                  out_hbm.at[pl.ds(base0 + koff * C, C)],
                                  wsems[s]).wait()

        stage(0, 0)
        fire(0)

        def pair(k2, carry):
            c0 = 2 * k2

            @pl.when(k2 >= 1)
            def _():
                wait_wb(c0 - 1, 1)

            stage(c0 + 1, 1)
            fire(1)
            wait_gathers(0)
            sum_wb(c0, 0)

            @pl.when(k2 + 1 < half)
            def _():
                wait_wb(c0, 0)
                stage(c0 + 2, 0)
                fire(0)

            wait_gathers(1)
            sum_wb(c0 + 1, 1)
            return carry

        lax.fori_loop(0, half, pair, 0)
        wait_wb(2 * half - 2, 0)
        wait_wb(2 * half - 1, 1)

    return pl.kernel(
        body,
        out_type=jax.ShapeDtypeStruct((P, 128), F32),
        mesh=mesh,
        scratch_types=[
            pltpu.VMEM((C,), jnp.int32),
            pltpu.VMEM((C,), jnp.int32),
            pltpu.VMEM((C,), jnp.int32),
            pltpu.VMEM((C,), jnp.int32),
            pltpu.VMEM((C,), jnp.int32),
            pltpu.VMEM((C,), jnp.int32),
            pltpu.VMEM((2, 3, C, 128), F32),
            pltpu.SemaphoreType.DMA,
            pltpu.SemaphoreType.DMA,
            pltpu.SemaphoreType.DMA,
            pltpu.SemaphoreType.DMA,
        ],
    )


def _local_potentials(f_bond_p, WlT, BM):
    EP, K = f_bond_p.shape
    H = WlT.shape[1]

    def body(fb_ref, w_ref, lp_ref, msg_ref):
        lp = jnp.dot(fb_ref[...], w_ref[...], preferred_element_type=F32)
        lp_ref[...] = lp
        msg_ref[...] = jnp.maximum(lp, 0.0)

    return pl.pallas_call(
        body,
        grid=(EP // BM,),
        in_specs=[
            pl.BlockSpec((BM, K), lambda i: (i, 0)),
            pl.BlockSpec((K, H), lambda i: (0, 0)),
        ],
        out_specs=[
            pl.BlockSpec((BM, H), lambda i: (i, 0)),
            pl.BlockSpec((BM, H), lambda i: (i, 0)),
        ],
        out_shape=[
            jax.ShapeDtypeStruct((EP, H), F32),
            jax.ShapeDtypeStruct((EP, H), F32),
        ],
    )(f_bond_p, WlT)


def _gru_update(sn, lp, msg, WmT, WihT, WhhT, bih, bhh, BM):
    EP, H = sn.shape

    def body(sn_ref, lp_ref, msg_ref, wm_ref, wi_ref, wh_ref, bi_ref, bh_ref, out_ref):
        nb = jnp.dot(sn_ref[...], wm_ref[...], preferred_element_type=F32)
        new = jnp.maximum(lp_ref[...] + nb, 0.0)
        h = msg_ref[...]
        gi = jnp.dot(new, wi_ref[...], preferred_element_type=F32) + bi_ref[...]
        gh = jnp.dot(h, wh_ref[...], preferred_element_type=F32) + bh_ref[...]
        r = jax.nn.sigmoid(gi[:, :H] + gh[:, :H])
        z = jax.nn.sigmoid(gi[:, H:2 * H] + gh[:, H:2 * H])
        n = jnp.tanh(gi[:, 2 * H:] + r * gh[:, 2 * H:])
        out_ref[...] = (1.0 - z) * n + z * h

        @pl.when(pl.program_id(0) == 0)
        def _():
            out_ref[0:1, :] = jnp.zeros((1, H), F32)

    return pl.pallas_call(
        body,
        grid=(EP // BM,),
        in_specs=[
            pl.BlockSpec((BM, H), lambda i: (i, 0)),
            pl.BlockSpec((BM, H), lambda i: (i, 0)),
            pl.BlockSpec((BM, H), lambda i: (i, 0)),
            pl.BlockSpec((H, H), lambda i: (0, 0)),
            pl.BlockSpec((H, 3 * H), lambda i: (0, 0)),
            pl.BlockSpec((H, 3 * H), lambda i: (0, 0)),
            pl.BlockSpec((1, 3 * H), lambda i: (0, 0)),
            pl.BlockSpec((1, 3 * H), lambda i: (0, 0)),
        ],
        out_specs=pl.BlockSpec((BM, H), lambda i: (i, 0)),
        out_shape=jax.ShapeDtypeStruct((EP, H), F32),
    )(sn, lp, msg, WmT, WihT, WhhT, bih, bhh)


def _node_embedding(f_nuc_p, nnm, W1T, W2T, BM):
    NP, K = f_nuc_p.shape
    H = W2T.shape[1]

    def body(fn_ref, nm_ref, w1_ref, w2_ref, out_ref):
        acc = jnp.dot(fn_ref[...], w1_ref[...], preferred_element_type=F32)
        acc = acc + jnp.dot(nm_ref[...], w2_ref[...], preferred_element_type=F32)
        out_ref[...] = jnp.maximum(acc, 0.0)

    return pl.pallas_call(
        body,
        grid=(NP // BM,),
        in_specs=[
            pl.BlockSpec((BM, K), lambda i: (i, 0)),
            pl.BlockSpec((BM, H), lambda i: (i, 0)),
            pl.BlockSpec((K, H), lambda i: (0, 0)),
            pl.BlockSpec((H, H), lambda i: (0, 0)),
        ],
        out_specs=pl.BlockSpec((BM, H), lambda i: (i, 0)),
        out_shape=jax.ShapeDtypeStruct((NP, H), F32),
    )(f_nuc_p, nnm, W1T, W2T)


def _bilstm_maxpool(ne_t, WifT, WhfT, bf, WibT, WhbT, bb, T_b):
    Lq, Bq, H = ne_t.shape
    HH = WhfT.shape[0]
    G = Lq // T_b

    def body(nef_ref, neb_ref, wif_ref, whf_ref, bf_ref, wib_ref, whb_ref, bb_ref,
             out_ref, hf_s, cf_s, hb_s, cb_s, mf_s, mb_s):
        i = pl.program_id(0)

        @pl.when(i == 0)
        def _():
            z = jnp.zeros((Bq, HH), F32)
            hf_s[...] = z
            cf_s[...] = z
            hb_s[...] = z
            cb_s[...] = z
            m0 = jnp.full((Bq, HH), -jnp.inf, F32)
            mf_s[...] = m0
            mb_s[...] = m0

        def one_dir(x, h, c, wi_ref, wh_ref, b_ref):
            g = (jnp.dot(x, wi_ref[...], preferred_element_type=F32)
                 + jnp.dot(h, wh_ref[...], preferred_element_type=F32)
                 + b_ref[...])
            ig = jax.nn.sigmoid(g[:, :HH])
            fg = jax.nn.sigmoid(g[:, HH:2 * HH])
            gg = jnp.tanh(g[:, 2 * HH:3 * HH])
            og = jax.nn.sigmoid(g[:, 3 * HH:])
            c = fg * c + ig * gg
            h = og * jnp.tanh(c)
            return h, c

        def step(tt, carry):
            hf, cf, hb, cb, mf, mb = carry
            hf, cf = one_dir(nef_ref[tt], hf, cf, wif_ref, whf_ref, bf_ref)
            mf = jnp.maximum(mf, hf)
            hb, cb = one_dir(neb_ref[T_b - 1 - tt], hb, cb, wib_ref, whb_ref, bb_ref)
            mb = jnp.maximum(mb, hb)
            return hf, cf, hb, cb, mf, mb

        init = (hf_s[...], cf_s[...], hb_s[...], cb_s[...], mf_s[...], mb_s[...])
        hf, cf, hb, cb, mf, mb = lax.fori_loop(0, T_b, step, init)
        hf_s[...] = hf
        cf_s[...] = cf
        hb_s[...] = hb
        cb_s[...] = cb
        mf_s[...] = mf
        mb_s[...] = mb

        @pl.when(i == G - 1)
        def _():
            out_ref[...] = jnp.concatenate([mf, mb], axis=1)

    return pl.pallas_call(
        body,
        grid=(G,),
        in_specs=[
            pl.BlockSpec((T_b, Bq, H), lambda i: (i, 0, 0)),
            pl.BlockSpec((T_b, Bq, H), lambda i: (G - 1 - i, 0, 0)),
            pl.BlockSpec((H, 4 * HH), lambda i: (0, 0)),
            pl.BlockSpec((HH, 4 * HH), lambda i: (0, 0)),
            pl.BlockSpec((1, 4 * HH), lambda i: (0, 0)),
            pl.BlockSpec((H, 4 * HH), lambda i: (0, 0)),
            pl.BlockSpec((HH, 4 * HH), lambda i: (0, 0)),
            pl.BlockSpec((1, 4 * HH), lambda i: (0, 0)),
        ],
        out_specs=pl.BlockSpec((Bq, 2 * HH), lambda i: (0, 0)),
        out_shape=jax.ShapeDtypeStruct((Bq, 2 * HH), F32),
        scratch_shapes=[pltpu.VMEM((Bq, HH), F32)] * 6,
    )(ne_t, ne_t, WifT, WhfT, bf, WibT, WhbT, bb)


def _pad_rows(x, P):
    n = x.shape[0]
    if n == P:
        return x
    return jnp.concatenate(
        [x, jnp.zeros((P - n,) + x.shape[1:], x.dtype)], axis=0)


def kernel(f_nuc, f_bond, node_graph, message_graph, all_bonds, scope,
           W_local, W_msg, W_node_emb,
           gru_w_ih, gru_w_hh, gru_b_ih, gru_b_hh,
           lstm_w_ih_f, lstm_w_hh_f, lstm_b_ih_f, lstm_b_hh_f,
           lstm_w_ih_b, lstm_w_hh_b, lstm_b_ih_b, lstm_b_hh_b):
    E = f_bond.shape[0]
    N = f_nuc.shape[0]
    H = W_msg.shape[0]
    B = scope.shape[0]
    L = N // B
    HH = lstm_w_hh_f.shape[1]
    NC, NS = _sc_info()
    NW = NC * NS
    C = 128
    unit = NW * C * 2

    EP = ((E + unit - 1) // unit) * unit
    NP = ((N + unit - 1) // unit) * unit
    e_rows = EP // NW
    n_rows = NP // NW

    f_bond_p = _pad_rows(f_bond.astype(F32), EP)
    f_nuc_p = _pad_rows(f_nuc.astype(F32), NP)
    mg = _pad_rows(message_graph.astype(jnp.int32), EP)
    ng = _pad_rows(node_graph.astype(jnp.int32), NP)
    i0, i1, i2 = mg[:, 0], mg[:, 1], mg[:, 2]
    n0, n1, n2 = ng[:, 0], ng[:, 1], ng[:, 2]

    WlT = W_local.T.astype(F32)
    WmT = W_msg.T
    WihT = gru_w_ih.T
    WhhT = gru_w_hh.T
    bih = gru_b_ih.reshape(1, 3 * H)
    bhh = gru_b_hh.reshape(1, 3 * H)
    W1T = W_node_emb[:, :4].T
    W2T = W_node_emb[:, 4:].T
    WifT = lstm_w_ih_f.T
    WhfT = lstm_w_hh_f.T
    bf = (lstm_b_ih_f + lstm_b_hh_f).reshape(1, 4 * HH)
    WibT = lstm_w_ih_b.T
    WhbT = lstm_w_hh_b.T
    bb = (lstm_b_ih_b + lstm_b_hh_b).reshape(1, 4 * HH)

    BM = 2048
    lp, msgs = _local_potentials(f_bond_p, WlT, BM)

    gsum_e = _build_gather_sum(EP, e_rows, e_rows // C, C, NC, NS)
    for _ in range(2):
        sn = gsum_e(msgs, i0, i1, i2)
        msgs = _gru_update(sn, lp, msgs, WmT, WihT, WhhT, bih, bhh, BM)

    gsum_n = _build_gather_sum(NP, n_rows, n_rows // C, C, NC, NS)
    nnm = gsum_n(msgs, n0, n1, n2)

    ne = _node_embedding(f_nuc_p, nnm, W1T, W2T, BM)
    ne_t = ne[:N].reshape(B, L, H).transpose(1, 0, 2)

    T_b = 1
    for d in range(min(25, L), 0, -1):
        if L % d == 0:
            T_b = d
            break
    rep = _bilstm_maxpool(ne_t, WifT, WhfT, bf, WibT, WhbT, bb, T_b)
    return rep


# speedup vs baseline: 1.7893x; 1.7856x over previous
import functools

import jax
import jax.numpy as jnp
from jax import lax
from jax.experimental import pallas as pl
from jax.experimental.pallas import tpu as pltpu
from jax.experimental.pallas import tpu_sc as plsc

F32 = jnp.float32


def _sc_info():
    try:
        info = plsc.get_sparse_core_info()
        return info.num_cores, info.num_subcores
    except Exception:
        return 2, 16


def _build_gather_sum(P, n_rows, n_chunks, C, NC, NS):
    mesh = plsc.VectorSubcoreMesh(core_axis_name="c", subcore_axis_name="s")

    def body(msgs_hbm, i0_hbm, i1_hbm, i2_hbm, out_hbm,
             ia0, ia1, ia2, ib0, ib1, ib2, rows_v, gsem0, gsem1, wsem0, wsem1):
        wid = lax.axis_index("s") * NC + lax.axis_index("c")
        base0 = wid * n_rows
        idx_hbms = (i0_hbm, i1_hbm, i2_hbm)
        idx_vs = ((ia0, ia1, ia2), (ib0, ib1, ib2))
        gsems = (gsem0, gsem1)
        wsems = (wsem0, wsem1)

        def stage_fire(koff, s):
            base = base0 + koff * C
            for j in range(3):
                pltpu.sync_copy(idx_hbms[j].at[pl.ds(base, C)], idx_vs[s][j])
            return [pltpu.async_copy(msgs_hbm.at[idx_vs[s][j]],
                                     rows_v.at[s, j], gsems[s])
                    for j in range(3)]

        def sum_wb(koff, s):
            def row(r, c2):
                for l in range(8):
                    sl = pl.ds(l * 16, 16)
                    rows_v[s, 0, r, sl] = (rows_v[s, 0, r, sl]
                                           + rows_v[s, 1, r, sl]
                                           + rows_v[s, 2, r, sl])
                return c2

            lax.fori_loop(0, C, row, 0)
            return pltpu.async_copy(rows_v.at[s, 0],
                                    out_hbm.at[pl.ds(base0 + koff * C, C)],
                                    wsems[s])

        gh = [None] * n_chunks
        wbh = [None] * n_chunks
        gh[0] = stage_fire(0, 0)
        for k in range(n_chunks):
            s = k % 2
            if k + 1 < n_chunks:
                if k >= 1:
                    wbh[k - 1].wait()
                gh[k + 1] = stage_fire(k + 1, 1 - s)
            for h in gh[k]:
                h.wait()
            wbh[k] = sum_wb(k, s)
        if n_chunks >= 2:
            wbh[n_chunks - 2].wait()
        wbh[n_chunks - 1].wait()

    return pl.kernel(
        body,
        out_type=jax.ShapeDtypeStruct((P, 128), F32),
        mesh=mesh,
        scratch_types=[
            pltpu.VMEM((C,), jnp.int32),
            pltpu.VMEM((C,), jnp.int32),
            pltpu.VMEM((C,), jnp.int32),
            pltpu.VMEM((C,), jnp.int32),
            pltpu.VMEM((C,), jnp.int32),
            pltpu.VMEM((C,), jnp.int32),
            pltpu.VMEM((2, 3, C, 128), F32),
            pltpu.SemaphoreType.DMA,
            pltpu.SemaphoreType.DMA,
            pltpu.SemaphoreType.DMA,
            pltpu.SemaphoreType.DMA,
        ],
    )


def _local_potentials(f_bond_p, WlT, BM):
    EP, K = f_bond_p.shape
    H = WlT.shape[1]

    def body(fb_ref, w_ref, lp_ref, msg_ref):
        lp = jnp.dot(fb_ref[...], w_ref[...], preferred_element_type=F32)
        lp_ref[...] = lp
        msg_ref[...] = jnp.maximum(lp, 0.0)

    return pl.pallas_call(
        body,
        grid=(EP // BM,),
        in_specs=[
            pl.BlockSpec((BM, K), lambda i: (i, 0)),
            pl.BlockSpec((K, H), lambda i: (0, 0)),
        ],
        out_specs=[
            pl.BlockSpec((BM, H), lambda i: (i, 0)),
            pl.BlockSpec((BM, H), lambda i: (i, 0)),
        ],
        out_shape=[
            jax.ShapeDtypeStruct((EP, H), F32),
            jax.ShapeDtypeStruct((EP, H), F32),
        ],
    )(f_bond_p, WlT)


def _gru_update(sn, lp, msg, WmT, WihT, WhhT, bih, bhh, BM):
    EP, H = sn.shape

    def body(sn_ref, lp_ref, msg_ref, wm_ref, wi_ref, wh_ref, bi_ref, bh_ref, out_ref):
        nb = jnp.dot(sn_ref[...], wm_ref[...], preferred_element_type=F32)
        new = jnp.maximum(lp_ref[...] + nb, 0.0)
        h = msg_ref[...]
        gi = jnp.dot(new, wi_ref[...], preferred_element_type=F32) + bi_ref[...]
        gh = jnp.dot(h, wh_ref[...], preferred_element_type=F32) + bh_ref[...]
        r = jax.nn.sigmoid(gi[:, :H] + gh[:, :H])
        z = jax.nn.sigmoid(gi[:, H:2 * H] + gh[:, H:2 * H])
        n = jnp.tanh(gi[:, 2 * H:] + r * gh[:, 2 * H:])
        out_ref[...] = (1.0 - z) * n + z * h

        @pl.when(pl.program_id(0) == 0)
        def _():
            out_ref[0:1, :] = jnp.zeros((1, H), F32)

    return pl.pallas_call(
        body,
        grid=(EP // BM,),
        in_specs=[
            pl.BlockSpec((BM, H), lambda i: (i, 0)),
            pl.BlockSpec((BM, H), lambda i: (i, 0)),
            pl.BlockSpec((BM, H), lambda i: (i, 0)),
            pl.BlockSpec((H, H), lambda i: (0, 0)),
            pl.BlockSpec((H, 3 * H), lambda i: (0, 0)),
            pl.BlockSpec((H, 3 * H), lambda i: (0, 0)),
            pl.BlockSpec((1, 3 * H), lambda i: (0, 0)),
            pl.BlockSpec((1, 3 * H), lambda i: (0, 0)),
        ],
        out_specs=pl.BlockSpec((BM, H), lambda i: (i, 0)),
        out_shape=jax.ShapeDtypeStruct((EP, H), F32),
    )(sn, lp, msg, WmT, WihT, WhhT, bih, bhh)


def _node_embedding(f_nuc_p, nnm, W1T, W2T, BM):
    NP, K = f_nuc_p.shape
    H = W2T.shape[1]

    def body(fn_ref, nm_ref, w1_ref, w2_ref, out_ref):
        acc = jnp.dot(fn_ref[...], w1_ref[...], preferred_element_type=F32)
        acc = acc + jnp.dot(nm_ref[...], w2_ref[...], preferred_element_type=F32)
        out_ref[...] = jnp.maximum(acc, 0.0)

    return pl.pallas_call(
        body,
        grid=(NP // BM,),
        in_specs=[
            pl.BlockSpec((BM, K), lambda i: (i, 0)),
            pl.BlockSpec((BM, H), lambda i: (i, 0)),
            pl.BlockSpec((K, H), lambda i: (0, 0)),
            pl.BlockSpec((H, H), lambda i: (0, 0)),
        ],
        out_specs=pl.BlockSpec((BM, H), lambda i: (i, 0)),
        out_shape=jax.ShapeDtypeStruct((NP, H), F32),
    )(f_nuc_p, nnm, W1T, W2T)


def _bilstm_maxpool(ne_t, WifT, WhfT, bf, WibT, WhbT, bb, T_b):
    Lq, Bq, H = ne_t.shape
    HH = WhfT.shape[0]
    G = Lq // T_b

    def body(nef_ref, neb_ref, wif_ref, whf_ref, bf_ref, wib_ref, whb_ref, bb_ref,
             out_ref, hf_s, cf_s, hb_s, cb_s, mf_s, mb_s):
        i = pl.program_id(0)

        @pl.when(i == 0)
        def _():
            z = jnp.zeros((Bq, HH), F32)
            hf_s[...] = z
            cf_s[...] = z
            hb_s[...] = z
            cb_s[...] = z
            m0 = jnp.full((Bq, HH), -jnp.inf, F32)
            mf_s[...] = m0
            mb_s[...] = m0

        def one_dir(x, h, c, wi_ref, wh_ref, b_ref):
            g = (jnp.dot(x, wi_ref[...], preferred_element_type=F32)
                 + jnp.dot(h, wh_ref[...], preferred_element_type=F32)
                 + b_ref[...])
            ig = jax.nn.sigmoid(g[:, :HH])
            fg = jax.nn.sigmoid(g[:, HH:2 * HH])
            gg = jnp.tanh(g[:, 2 * HH:3 * HH])
            og = jax.nn.sigmoid(g[:, 3 * HH:])
            c = fg * c + ig * gg
            h = og * jnp.tanh(c)
            return h, c

        def step(tt, carry):
            hf, cf, hb, cb, mf, mb = carry
            hf, cf = one_dir(nef_ref[tt], hf, cf, wif_ref, whf_ref, bf_ref)
            mf = jnp.maximum(mf, hf)
            hb, cb = one_dir(neb_ref[T_b - 1 - tt], hb, cb, wib_ref, whb_ref, bb_ref)
            mb = jnp.maximum(mb, hb)
            return hf, cf, hb, cb, mf, mb

        init = (hf_s[...], cf_s[...], hb_s[...], cb_s[...], mf_s[...], mb_s[...])
        hf, cf, hb, cb, mf, mb = lax.fori_loop(0, T_b, step, init)
        hf_s[...] = hf
        cf_s[...] = cf
        hb_s[...] = hb
        cb_s[...] = cb
        mf_s[...] = mf
        mb_s[...] = mb

        @pl.when(i == G - 1)
        def _():
            out_ref[...] = jnp.concatenate([mf, mb], axis=1)

    return pl.pallas_call(
        body,
        grid=(G,),
        in_specs=[
            pl.BlockSpec((T_b, Bq, H), lambda i: (i, 0, 0)),
            pl.BlockSpec((T_b, Bq, H), lambda i: (G - 1 - i, 0, 0)),
            pl.BlockSpec((H, 4 * HH), lambda i: (0, 0)),
            pl.BlockSpec((HH, 4 * HH), lambda i: (0, 0)),
            pl.BlockSpec((1, 4 * HH), lambda i: (0, 0)),
            pl.BlockSpec((H, 4 * HH), lambda i: (0, 0)),
            pl.BlockSpec((HH, 4 * HH), lambda i: (0, 0)),
            pl.BlockSpec((1, 4 * HH), lambda i: (0, 0)),
        ],
        out_specs=pl.BlockSpec((Bq, 2 * HH), lambda i: (0, 0)),
        out_shape=jax.ShapeDtypeStruct((Bq, 2 * HH), F32),
        scratch_shapes=[pltpu.VMEM((Bq, HH), F32)] * 6,
    )(ne_t, ne_t, WifT, WhfT, bf, WibT, WhbT, bb)


def _pad_rows(x, P):
    n = x.shape[0]
    if n == P:
        return x
    return jnp.concatenate(
        [x, jnp.zeros((P - n,) + x.shape[1:], x.dtype)], axis=0)


def kernel(f_nuc, f_bond, node_graph, message_graph, all_bonds, scope,
           W_local, W_msg, W_node_emb,
           gru_w_ih, gru_w_hh, gru_b_ih, gru_b_hh,
           lstm_w_ih_f, lstm_w_hh_f, lstm_b_ih_f, lstm_b_hh_f,
           lstm_w_ih_b, lstm_w_hh_b, lstm_b_ih_b, lstm_b_hh_b):
    E = f_bond.shape[0]
    N = f_nuc.shape[0]
    H = W_msg.shape[0]
    B = scope.shape[0]
    L = N // B
    HH = lstm_w_hh_f.shape[1]
    NC, NS = _sc_info()
    NW = NC * NS
    C = 128
    unit = NW * C

    EP = ((E + unit - 1) // unit) * unit
    NP = ((N + unit - 1) // unit) * unit
    e_rows = EP // NW
    n_rows = NP // NW

    f_bond_p = _pad_rows(f_bond.astype(F32), EP)
    f_nuc_p = _pad_rows(f_nuc.astype(F32), NP)
    mg = _pad_rows(message_graph.astype(jnp.int32), EP)
    ng = _pad_rows(node_graph.astype(jnp.int32), NP)
    i0, i1, i2 = mg[:, 0], mg[:, 1], mg[:, 2]
    n0, n1, n2 = ng[:, 0], ng[:, 1], ng[:, 2]

    WlT = W_local.T.astype(F32)
    WmT = W_msg.T
    WihT = gru_w_ih.T
    WhhT = gru_w_hh.T
    bih = gru_b_ih.reshape(1, 3 * H)
    bhh = gru_b_hh.reshape(1, 3 * H)
    W1T = W_node_emb[:, :4].T
    W2T = W_node_emb[:, 4:].T
    WifT = lstm_w_ih_f.T
    WhfT = lstm_w_hh_f.T
    bf = (lstm_b_ih_f + lstm_b_hh_f).reshape(1, 4 * HH)
    WibT = lstm_w_ih_b.T
    WhbT = lstm_w_hh_b.T
    bb = (lstm_b_ih_b + lstm_b_hh_b).reshape(1, 4 * HH)

    BM = 2048
    lp, msgs = _local_potentials(f_bond_p, WlT, BM)

    gsum_e = _build_gather_sum(EP, e_rows, e_rows // C, C, NC, NS)
    for _ in range(2):
        sn = gsum_e(msgs, i0, i1, i2)
        msgs = _gru_update(sn, lp, msgs, WmT, WihT, WhhT, bih, bhh, BM)

    gsum_n = _build_gather_sum(NP, n_rows, n_rows // C, C, NC, NS)
    nnm = gsum_n(msgs, n0, n1, n2)

    ne = _node_embedding(f_nuc_p, nnm, W1T, W2T, BM)
    ne_t = ne[:N].reshape(B, L, H).transpose(1, 0, 2)

    T_b = 1
    for d in range(min(25, L), 0, -1):
        if L % d == 0:
            T_b = d
            break
    rep = _bilstm_maxpool(ne_t, WifT, WhfT, bf, WibT, WhbT, bb, T_b)
    return rep


# speedup vs baseline: 3.1930x; 1.7846x over previous
import functools

import jax
import jax.numpy as jnp
from jax import lax
from jax.experimental import pallas as pl
from jax.experimental.pallas import tpu as pltpu
from jax.experimental.pallas import tpu_sc as plsc

F32 = jnp.float32


def _sc_info():
    try:
        info = plsc.get_sparse_core_info()
        return info.num_cores, info.num_subcores
    except Exception:
        return 2, 16


def _build_gather_sum(P, n_rows, n_chunks, C, NC, NS):
    mesh = plsc.VectorSubcoreMesh(core_axis_name="c", subcore_axis_name="s")

    def body(msgs_hbm, i0_hbm, i1_hbm, i2_hbm, out_hbm,
             ia0, ia1, ia2, ib0, ib1, ib2, rows_v, gsem0, gsem1, wsem0, wsem1):
        wid = lax.axis_index("s") * NC + lax.axis_index("c")
        base0 = wid * n_rows
        idx_hbms = (i0_hbm, i1_hbm, i2_hbm)
        idx_vs = ((ia0, ia1, ia2), (ib0, ib1, ib2))
        gsems = (gsem0, gsem1)
        wsems = (wsem0, wsem1)

        def stage_fire(koff, s):
            base = base0 + koff * C
            for j in range(3):
                pltpu.sync_copy(idx_hbms[j].at[pl.ds(base, C)], idx_vs[s][j])
            return [pltpu.async_copy(msgs_hbm.at[idx_vs[s][j]],
                                     rows_v.at[s, j], gsems[s])
                    for j in range(3)]

        def sum_wb(koff, s):
            def row(r, c2):
                for l in range(8):
                    sl = pl.ds(l * 16, 16)
                    rows_v[s, 0, r, sl] = (rows_v[s, 0, r, sl]
                                           + rows_v[s, 1, r, sl]
                                           + rows_v[s, 2, r, sl])
                return c2

            lax.fori_loop(0, C, row, 0)
            return pltpu.async_copy(rows_v.at[s, 0],
                                    out_hbm.at[pl.ds(base0 + koff * C, C)],
                                    wsems[s])

        gh = [None] * n_chunks
        wbh = [None] * n_chunks
        gh[0] = stage_fire(0, 0)
        for k in range(n_chunks):
            s = k % 2
            if k + 1 < n_chunks:
                if k >= 1:
                    wbh[k - 1].wait()
                gh[k + 1] = stage_fire(k + 1, 1 - s)
            for h in gh[k]:
                h.wait()
            wbh[k] = sum_wb(k, s)
        if n_chunks >= 2:
            wbh[n_chunks - 2].wait()
        wbh[n_chunks - 1].wait()

    return pl.kernel(
        body,
        out_type=jax.ShapeDtypeStruct((P, 128), F32),
        mesh=mesh,
        scratch_types=[
            pltpu.VMEM((C,), jnp.int32),
            pltpu.VMEM((C,), jnp.int32),
            pltpu.VMEM((C,), jnp.int32),
            pltpu.VMEM((C,), jnp.int32),
            pltpu.VMEM((C,), jnp.int32),
            pltpu.VMEM((C,), jnp.int32),
            pltpu.VMEM((2, 3, C, 128), F32),
            pltpu.SemaphoreType.DMA,
            pltpu.SemaphoreType.DMA,
            pltpu.SemaphoreType.DMA,
            pltpu.SemaphoreType.DMA,
        ],
    )


def _local_potentials(f_bond_p, WlT, BM):
    EP, K = f_bond_p.shape
    H = WlT.shape[1]

    def body(fb_ref, w_ref, lp_ref, msg_ref):
        lp = jnp.dot(fb_ref[...], w_ref[...], preferred_element_type=F32)
        lp_ref[...] = lp
        msg_ref[...] = jnp.maximum(lp, 0.0)

    return pl.pallas_call(
        body,
        grid=(EP // BM,),
        in_specs=[
            pl.BlockSpec((BM, K), lambda i: (i, 0)),
            pl.BlockSpec((K, H), lambda i: (0, 0)),
        ],
        out_specs=[
            pl.BlockSpec((BM, H), lambda i: (i, 0)),
            pl.BlockSpec((BM, H), lambda i: (i, 0)),
        ],
        out_shape=[
            jax.ShapeDtypeStruct((EP, H), F32),
            jax.ShapeDtypeStruct((EP, H), F32),
        ],
    )(f_bond_p, WlT)


def _gru_update(sn, lp, msg, WmT, WihT, WhhT, bih, bhh, BM):
    EP, H = sn.shape

    def body(sn_ref, lp_ref, msg_ref, wm_ref, wi_ref, wh_ref, bi_ref, bh_ref, out_ref):
        nb = jnp.dot(sn_ref[...], wm_ref[...], preferred_element_type=F32)
        new = jnp.maximum(lp_ref[...] + nb, 0.0)
        h = msg_ref[...]
        gi = jnp.dot(new, wi_ref[...], preferred_element_type=F32) + bi_ref[...]
        gh = jnp.dot(h, wh_ref[...], preferred_element_type=F32) + bh_ref[...]
        r = jax.nn.sigmoid(gi[:, :H] + gh[:, :H])
        z = jax.nn.sigmoid(gi[:, H:2 * H] + gh[:, H:2 * H])
        n = jnp.tanh(gi[:, 2 * H:] + r * gh[:, 2 * H:])
        out_ref[...] = (1.0 - z) * n + z * h

        @pl.when(pl.program_id(0) == 0)
        def _():
            out_ref[0:1, :] = jnp.zeros((1, H), F32)

    return pl.pallas_call(
        body,
        grid=(EP // BM,),
        in_specs=[
            pl.BlockSpec((BM, H), lambda i: (i, 0)),
            pl.BlockSpec((BM, H), lambda i: (i, 0)),
            pl.BlockSpec((BM, H), lambda i: (i, 0)),
            pl.BlockSpec((H, H), lambda i: (0, 0)),
            pl.BlockSpec((H, 3 * H), lambda i: (0, 0)),
            pl.BlockSpec((H, 3 * H), lambda i: (0, 0)),
            pl.BlockSpec((1, 3 * H), lambda i: (0, 0)),
            pl.BlockSpec((1, 3 * H), lambda i: (0, 0)),
        ],
        out_specs=pl.BlockSpec((BM, H), lambda i: (i, 0)),
        out_shape=jax.ShapeDtypeStruct((EP, H), F32),
    )(sn, lp, msg, WmT, WihT, WhhT, bih, bhh)


def _node_embedding(f_nuc_p, nnm, W1T, W2T, BM):
    NP, K = f_nuc_p.shape
    H = W2T.shape[1]

    def body(fn_ref, nm_ref, w1_ref, w2_ref, out_ref):
        acc = jnp.dot(fn_ref[...], w1_ref[...], preferred_element_type=F32)
        acc = acc + jnp.dot(nm_ref[...], w2_ref[...], preferred_element_type=F32)
        out_ref[...] = jnp.maximum(acc, 0.0)

    return pl.pallas_call(
        body,
        grid=(NP // BM,),
        in_specs=[
            pl.BlockSpec((BM, K), lambda i: (i, 0)),
            pl.BlockSpec((BM, H), lambda i: (i, 0)),
            pl.BlockSpec((K, H), lambda i: (0, 0)),
            pl.BlockSpec((H, H), lambda i: (0, 0)),
        ],
        out_specs=pl.BlockSpec((BM, H), lambda i: (i, 0)),
        out_shape=jax.ShapeDtypeStruct((NP, H), F32),
    )(f_nuc_p, nnm, W1T, W2T)


def _bilstm_maxpool(ne_t, WifT, WhfT, bf, WibT, WhbT, bb, T_b):
    Lq, Bq, H = ne_t.shape
    HH = WhfT.shape[0]
    G = Lq // T_b

    def body(nef_ref, neb_ref, wif_ref, whf_ref, bf_ref, wib_ref, whb_ref, bb_ref,
             out_ref, hf_s, cf_s, hb_s, cb_s, mf_s, mb_s):
        i = pl.program_id(0)

        @pl.when(i == 0)
        def _():
            z = jnp.zeros((Bq, HH), F32)
            hf_s[...] = z
            cf_s[...] = z
            hb_s[...] = z
            cb_s[...] = z
            m0 = jnp.full((Bq, HH), -jnp.inf, F32)
            mf_s[...] = m0
            mb_s[...] = m0

        def one_dir(x, h, c, wi_ref, wh_ref, b_ref):
            g = (jnp.dot(x, wi_ref[...], preferred_element_type=F32)
                 + jnp.dot(h, wh_ref[...], preferred_element_type=F32)
                 + b_ref[...])
            ig = jax.nn.sigmoid(g[:, :HH])
            fg = jax.nn.sigmoid(g[:, HH:2 * HH])
            gg = jnp.tanh(g[:, 2 * HH:3 * HH])
            og = jax.nn.sigmoid(g[:, 3 * HH:])
            c = fg * c + ig * gg
            h = og * jnp.tanh(c)
            return h, c

        def step(tt, carry):
            hf, cf, hb, cb, mf, mb = carry
            hf, cf = one_dir(nef_ref[tt], hf, cf, wif_ref, whf_ref, bf_ref)
            mf = jnp.maximum(mf, hf)
            hb, cb = one_dir(neb_ref[T_b - 1 - tt], hb, cb, wib_ref, whb_ref, bb_ref)
            mb = jnp.maximum(mb, hb)
            return hf, cf, hb, cb, mf, mb

        init = (hf_s[...], cf_s[...], hb_s[...], cb_s[...], mf_s[...], mb_s[...])
        hf, cf, hb, cb, mf, mb = lax.fori_loop(0, T_b, step, init)
        hf_s[...] = hf
        cf_s[...] = cf
        hb_s[...] = hb
        cb_s[...] = cb
        mf_s[...] = mf
        mb_s[...] = mb

        @pl.when(i == G - 1)
        def _():
            out_ref[...] = jnp.concatenate([mf, mb], axis=1)

    return pl.pallas_call(
        body,
        grid=(G,),
        in_specs=[
            pl.BlockSpec((T_b, Bq, H), lambda i: (i, 0, 0)),
            pl.BlockSpec((T_b, Bq, H), lambda i: (G - 1 - i, 0, 0)),
            pl.BlockSpec((H, 4 * HH), lambda i: (0, 0)),
            pl.BlockSpec((HH, 4 * HH), lambda i: (0, 0)),
            pl.BlockSpec((1, 4 * HH), lambda i: (0, 0)),
            pl.BlockSpec((H, 4 * HH), lambda i: (0, 0)),
            pl.BlockSpec((HH, 4 * HH), lambda i: (0, 0)),
            pl.BlockSpec((1, 4 * HH), lambda i: (0, 0)),
        ],
        out_specs=pl.BlockSpec((Bq, 2 * HH), lambda i: (0, 0)),
        out_shape=jax.ShapeDtypeStruct((Bq, 2 * HH), F32),
        scratch_shapes=[pltpu.VMEM((Bq, HH), F32)] * 6,
    )(ne_t, ne_t, WifT, WhfT, bf, WibT, WhbT, bb)


def _pad_rows(x, P):
    n = x.shape[0]
    if n == P:
        return x
    return jnp.concatenate(
        [x, jnp.zeros((P - n,) + x.shape[1:], x.dtype)], axis=0)


def kernel(f_nuc, f_bond, node_graph, message_graph, all_bonds, scope,
           W_local, W_msg, W_node_emb,
           gru_w_ih, gru_w_hh, gru_b_ih, gru_b_hh,
           lstm_w_ih_f, lstm_w_hh_f, lstm_b_ih_f, lstm_b_hh_f,
           lstm_w_ih_b, lstm_w_hh_b, lstm_b_ih_b, lstm_b_hh_b):
    E = f_bond.shape[0]
    N = f_nuc.shape[0]
    H = W_msg.shape[0]
    B = scope.shape[0]
    L = N // B
    HH = lstm_w_hh_f.shape[1]
    NC, NS = _sc_info()
    NW = NC * NS
    C = 128
    unit = NW * C

    EP = ((E + unit - 1) // unit) * unit
    NP = ((N + unit - 1) // unit) * unit
    e_rows = EP // NW
    n_rows = NP // NW

    f_bond_p = _pad_rows(f_bond.astype(F32), EP)
    f_nuc_p = _pad_rows(f_nuc.astype(F32), NP)
    def _pad_idx(x, P):
        n = x.shape[0]
        pad = (jnp.arange(P - n, dtype=jnp.int32) % E)[:, None]
        return jnp.concatenate(
            [x, jnp.broadcast_to(pad, (P - n, x.shape[1]))], axis=0)

    mg = _pad_idx(message_graph.astype(jnp.int32), EP)
    ng = _pad_idx(node_graph.astype(jnp.int32), NP)
    i0, i1, i2 = mg[:, 0], mg[:, 1], mg[:, 2]
    n0, n1, n2 = ng[:, 0], ng[:, 1], ng[:, 2]

    WlT = W_local.T.astype(F32)
    WmT = W_msg.T
    WihT = gru_w_ih.T
    WhhT = gru_w_hh.T
    bih = gru_b_ih.reshape(1, 3 * H)
    bhh = gru_b_hh.reshape(1, 3 * H)
    W1T = W_node_emb[:, :4].T
    W2T = W_node_emb[:, 4:].T
    WifT = lstm_w_ih_f.T
    WhfT = lstm_w_hh_f.T
    bf = (lstm_b_ih_f + lstm_b_hh_f).reshape(1, 4 * HH)
    WibT = lstm_w_ih_b.T
    WhbT = lstm_w_hh_b.T
    bb = (lstm_b_ih_b + lstm_b_hh_b).reshape(1, 4 * HH)

    BM = 2048
    lp, msgs = _local_potentials(f_bond_p, WlT, BM)

    gsum_e = _build_gather_sum(EP, e_rows, e_rows // C, C, NC, NS)
    for _ in range(2):
        sn = gsum_e(msgs, i0, i1, i2)
        msgs = _gru_update(sn, lp, msgs, WmT, WihT, WhhT, bih, bhh, BM)

    gsum_n = _build_gather_sum(NP, n_rows, n_rows // C, C, NC, NS)
    nnm = gsum_n(msgs, n0, n1, n2)

    ne = _node_embedding(f_nuc_p, nnm, W1T, W2T, BM)
    ne_t = ne[:N].reshape(B, L, H).transpose(1, 0, 2)

    T_b = 1
    for d in range(min(25, L), 0, -1):
        if L % d == 0:
            T_b = d
            break
    rep = _bilstm_maxpool(ne_t, WifT, WhfT, bf, WibT, WhbT, bb, T_b)
    return rep


# speedup vs baseline: 3.5133x; 1.1003x over previous
import functools

import jax
import jax.numpy as jnp
from jax import lax
from jax.experimental import pallas as pl
from jax.experimental.pallas import tpu as pltpu
from jax.experimental.pallas import tpu_sc as plsc

F32 = jnp.float32


def _sc_info():
    try:
        info = plsc.get_sparse_core_info()
        return info.num_cores, info.num_subcores
    except Exception:
        return 2, 16


def _build_gather_sum(P, n_rows, n_chunks, C, NC, NS, relu=False):
    mesh = plsc.VectorSubcoreMesh(core_axis_name="c", subcore_axis_name="s")

    def body(msgs_hbm, i0_hbm, i1_hbm, i2_hbm, out_hbm,
             ia0, ia1, ia2, ib0, ib1, ib2, rows_v, gsem0, gsem1, wsem0, wsem1):
        wid = lax.axis_index("s") * NC + lax.axis_index("c")
        base0 = wid * n_rows
        idx_hbms = (i0_hbm, i1_hbm, i2_hbm)
        idx_vs = ((ia0, ia1, ia2), (ib0, ib1, ib2))
        gsems = (gsem0, gsem1)
        wsems = (wsem0, wsem1)

        def stage_fire(koff, s):
            base = base0 + koff * C
            for j in range(3):
                pltpu.sync_copy(idx_hbms[j].at[pl.ds(base, C)], idx_vs[s][j])
            return [pltpu.async_copy(msgs_hbm.at[idx_vs[s][j]],
                                     rows_v.at[s, j], gsems[s])
                    for j in range(3)]

        def sum_wb(koff, s):
            def row(r, c2):
                for l in range(8):
                    sl = pl.ds(l * 16, 16)
                    if relu:
                        zero = jnp.zeros((16,), F32)
                        rows_v[s, 0, r, sl] = (
                            jnp.maximum(rows_v[s, 0, r, sl], zero)
                            + jnp.maximum(rows_v[s, 1, r, sl], zero)
                            + jnp.maximum(rows_v[s, 2, r, sl], zero))
                    else:
                        rows_v[s, 0, r, sl] = (rows_v[s, 0, r, sl]
                                               + rows_v[s, 1, r, sl]
                                               + rows_v[s, 2, r, sl])
                return c2

            lax.fori_loop(0, C, row, 0)
            return pltpu.async_copy(rows_v.at[s, 0],
                                    out_hbm.at[pl.ds(base0 + koff * C, C)],
                                    wsems[s])

        gh = [None] * n_chunks
        wbh = [None] * n_chunks
        gh[0] = stage_fire(0, 0)
        for k in range(n_chunks):
            s = k % 2
            if k + 1 < n_chunks:
                if k >= 1:
                    wbh[k - 1].wait()
                gh[k + 1] = stage_fire(k + 1, 1 - s)
            for h in gh[k]:
                h.wait()
            wbh[k] = sum_wb(k, s)
        if n_chunks >= 2:
            wbh[n_chunks - 2].wait()
        wbh[n_chunks - 1].wait()

    return pl.kernel(
        body,
        out_type=jax.ShapeDtypeStruct((P, 128), F32),
        mesh=mesh,
        scratch_types=[
            pltpu.VMEM((C,), jnp.int32),
            pltpu.VMEM((C,), jnp.int32),
            pltpu.VMEM((C,), jnp.int32),
            pltpu.VMEM((C,), jnp.int32),
            pltpu.VMEM((C,), jnp.int32),
            pltpu.VMEM((C,), jnp.int32),
            pltpu.VMEM((2, 3, C, 128), F32),
            pltpu.SemaphoreType.DMA,
            pltpu.SemaphoreType.DMA,
            pltpu.SemaphoreType.DMA,
            pltpu.SemaphoreType.DMA,
        ],
    )


def _local_potentials(f_bond, WlT, BM):
    E, K = f_bond.shape
    H = WlT.shape[1]

    def body(fb_ref, w_ref, lp_ref):
        lp_ref[...] = jnp.dot(fb_ref[...], w_ref[...],
                              preferred_element_type=F32)

    return pl.pallas_call(
        body,
        grid=(pl.cdiv(E, BM),),
        in_specs=[
            pl.BlockSpec((BM, K), lambda i: (i, 0)),
            pl.BlockSpec((K, H), lambda i: (0, 0)),
        ],
        out_specs=pl.BlockSpec((BM, H), lambda i: (i, 0)),
        out_shape=jax.ShapeDtypeStruct((E, H), F32),
    )(f_bond, WlT)


def _gru_update(sn, lp, msg, WmT, WihT, WhhT, bih, bhh, BM):
    E, H = lp.shape
    first = msg is None

    def body(sn_ref, lp_ref, *rest):
        if first:
            (wm_ref, wi_ref, wh_ref, bi_ref, bh_ref, out_ref) = rest
            h = jnp.maximum(lp_ref[...], 0.0)
        else:
            (msg_ref, wm_ref, wi_ref, wh_ref, bi_ref, bh_ref, out_ref) = rest
            h = msg_ref[...]
        nb = jnp.dot(sn_ref[...], wm_ref[...], preferred_element_type=F32)
        new = jnp.maximum(lp_ref[...] + nb, 0.0)
        gi = jnp.dot(new, wi_ref[...], preferred_element_type=F32) + bi_ref[...]
        gh = jnp.dot(h, wh_ref[...], preferred_element_type=F32) + bh_ref[...]
        r = jax.nn.sigmoid(gi[:, :H] + gh[:, :H])
        z = jax.nn.sigmoid(gi[:, H:2 * H] + gh[:, H:2 * H])
        n = jnp.tanh(gi[:, 2 * H:] + r * gh[:, 2 * H:])
        out_ref[...] = (1.0 - z) * n + z * h

        @pl.when(pl.program_id(0) == 0)
        def _():
            out_ref[0:1, :] = jnp.zeros((1, H), F32)

    blk = [pl.BlockSpec((BM, H), lambda i: (i, 0))]
    wspecs = [
        pl.BlockSpec((H, H), lambda i: (0, 0)),
        pl.BlockSpec((H, 3 * H), lambda i: (0, 0)),
        pl.BlockSpec((H, 3 * H), lambda i: (0, 0)),
        pl.BlockSpec((1, 3 * H), lambda i: (0, 0)),
        pl.BlockSpec((1, 3 * H), lambda i: (0, 0)),
    ]
    ins = (sn, lp) if first else (sn, lp, msg)
    return pl.pallas_call(
        body,
        grid=(pl.cdiv(E, BM),),
        in_specs=blk * len(ins) + wspecs,
        out_specs=pl.BlockSpec((BM, H), lambda i: (i, 0)),
        out_shape=jax.ShapeDtypeStruct((E, H), F32),
    )(*ins, WmT, WihT, WhhT, bih, bhh)


def _node_embedding(f_nuc_p, nnm, W1T, W2T, BM):
    NP, K = f_nuc_p.shape
    H = W2T.shape[1]

    def body(fn_ref, nm_ref, w1_ref, w2_ref, out_ref):
        acc = jnp.dot(fn_ref[...], w1_ref[...], preferred_element_type=F32)
        acc = acc + jnp.dot(nm_ref[...], w2_ref[...], preferred_element_type=F32)
        out_ref[...] = jnp.maximum(acc, 0.0)

    return pl.pallas_call(
        body,
        grid=(pl.cdiv(NP, BM),),
        in_specs=[
            pl.BlockSpec((BM, K), lambda i: (i, 0)),
            pl.BlockSpec((BM, H), lambda i: (i, 0)),
            pl.BlockSpec((K, H), lambda i: (0, 0)),
            pl.BlockSpec((H, H), lambda i: (0, 0)),
        ],
        out_specs=pl.BlockSpec((BM, H), lambda i: (i, 0)),
        out_shape=jax.ShapeDtypeStruct((NP, H), F32),
    )(f_nuc_p, nnm, W1T, W2T)


def _bilstm_maxpool(ne_t, WifT, WhfT, bf, WibT, WhbT, bb, T_b):
    Lq, Bq, H = ne_t.shape
    HH = WhfT.shape[0]
    G = Lq // T_b

    def body(nef_ref, neb_ref, wif_ref, whf_ref, bf_ref, wib_ref, whb_ref, bb_ref,
             out_ref, hf_s, cf_s, hb_s, cb_s, mf_s, mb_s):
        i = pl.program_id(0)

        @pl.when(i == 0)
        def _():
            z = jnp.zeros((Bq, HH), F32)
            hf_s[...] = z
            cf_s[...] = z
            hb_s[...] = z
            cb_s[...] = z
            m0 = jnp.full((Bq, HH), -jnp.inf, F32)
            mf_s[...] = m0
            mb_s[...] = m0

        def one_dir(x, h, c, wi_ref, wh_ref, b_ref):
            g = (jnp.dot(x, wi_ref[...], preferred_element_type=F32)
                 + jnp.dot(h, wh_ref[...], preferred_element_type=F32)
                 + b_ref[...])
            ig = jax.nn.sigmoid(g[:, :HH])
            fg = jax.nn.sigmoid(g[:, HH:2 * HH])
            gg = jnp.tanh(g[:, 2 * HH:3 * HH])
            og = jax.nn.sigmoid(g[:, 3 * HH:])
            c = fg * c + ig * gg
            h = og * jnp.tanh(c)
            return h, c

        def step(tt, carry):
            hf, cf, hb, cb, mf, mb = carry
            hf, cf = one_dir(nef_ref[tt], hf, cf, wif_ref, whf_ref, bf_ref)
            mf = jnp.maximum(mf, hf)
            hb, cb = one_dir(neb_ref[T_b - 1 - tt], hb, cb, wib_ref, whb_ref, bb_ref)
            mb = jnp.maximum(mb, hb)
            return hf, cf, hb, cb, mf, mb

        init = (hf_s[...], cf_s[...], hb_s[...], cb_s[...], mf_s[...], mb_s[...])
        hf, cf, hb, cb, mf, mb = lax.fori_loop(0, T_b, step, init)
        hf_s[...] = hf
        cf_s[...] = cf
        hb_s[...] = hb
        cb_s[...] = cb
        mf_s[...] = mf
        mb_s[...] = mb

        @pl.when(i == G - 1)
        def _():
            out_ref[...] = jnp.concatenate([mf, mb], axis=1)

    return pl.pallas_call(
        body,
        grid=(G,),
        in_specs=[
            pl.BlockSpec((T_b, Bq, H), lambda i: (i, 0, 0)),
            pl.BlockSpec((T_b, Bq, H), lambda i: (G - 1 - i, 0, 0)),
            pl.BlockSpec((H, 4 * HH), lambda i: (0, 0)),
            pl.BlockSpec((HH, 4 * HH), lambda i: (0, 0)),
            pl.BlockSpec((1, 4 * HH), lambda i: (0, 0)),
            pl.BlockSpec((H, 4 * HH), lambda i: (0, 0)),
            pl.BlockSpec((HH, 4 * HH), lambda i: (0, 0)),
            pl.BlockSpec((1, 4 * HH), lambda i: (0, 0)),
        ],
        out_specs=pl.BlockSpec((Bq, 2 * HH), lambda i: (0, 0)),
        out_shape=jax.ShapeDtypeStruct((Bq, 2 * HH), F32),
        scratch_shapes=[pltpu.VMEM((Bq, HH), F32)] * 6,
    )(ne_t, ne_t, WifT, WhfT, bf, WibT, WhbT, bb)


def _pad_rows(x, P):
    n = x.shape[0]
    if n == P:
        return x
    return jnp.concatenate(
        [x, jnp.zeros((P - n,) + x.shape[1:], x.dtype)], axis=0)


def kernel(f_nuc, f_bond, node_graph, message_graph, all_bonds, scope,
           W_local, W_msg, W_node_emb,
           gru_w_ih, gru_w_hh, gru_b_ih, gru_b_hh,
           lstm_w_ih_f, lstm_w_hh_f, lstm_b_ih_f, lstm_b_hh_f,
           lstm_w_ih_b, lstm_w_hh_b, lstm_b_ih_b, lstm_b_hh_b):
    E = f_bond.shape[0]
    N = f_nuc.shape[0]
    H = W_msg.shape[0]
    B = scope.shape[0]
    L = N // B
    HH = lstm_w_hh_f.shape[1]
    NC, NS = _sc_info()
    NW = NC * NS
    C = 128
    unit = NW * C

    EP = ((E + unit - 1) // unit) * unit
    NP = ((N + unit - 1) // unit) * unit
    e_rows = EP // NW
    n_rows = NP // NW

    def _pad_idx(x, P):
        n = x.shape[0]
        pad = (jnp.arange(P - n, dtype=jnp.int32) % E)[:, None]
        return jnp.concatenate(
            [x, jnp.broadcast_to(pad, (P - n, x.shape[1]))], axis=0)

    mg = _pad_idx(message_graph.astype(jnp.int32), EP)
    ng = _pad_idx(node_graph.astype(jnp.int32), NP)
    i0, i1, i2 = mg[:, 0], mg[:, 1], mg[:, 2]
    n0, n1, n2 = ng[:, 0], ng[:, 1], ng[:, 2]

    WlT = W_local.T.astype(F32)
    WmT = W_msg.T
    WihT = gru_w_ih.T
    WhhT = gru_w_hh.T
    bih = gru_b_ih.reshape(1, 3 * H)
    bhh = gru_b_hh.reshape(1, 3 * H)
    W1T = W_node_emb[:, :4].T
    W2T = W_node_emb[:, 4:].T
    WifT = lstm_w_ih_f.T
    WhfT = lstm_w_hh_f.T
    bf = (lstm_b_ih_f + lstm_b_hh_f).reshape(1, 4 * HH)
    WibT = lstm_w_ih_b.T
    WhbT = lstm_w_hh_b.T
    bb = (lstm_b_ih_b + lstm_b_hh_b).reshape(1, 4 * HH)

    BM = 2048
    lp = _local_potentials(f_bond.astype(F32), WlT, BM)

    gsum_e_relu = _build_gather_sum(EP, e_rows, e_rows // C, C, NC, NS, relu=True)
    gsum_e = _build_gather_sum(EP, e_rows, e_rows // C, C, NC, NS)
    sn = gsum_e_relu(lp, i0, i1, i2)
    msgs = _gru_update(sn, lp, None, WmT, WihT, WhhT, bih, bhh, BM)
    sn = gsum_e(msgs, i0, i1, i2)
    msgs = _gru_update(sn, lp, msgs, WmT, WihT, WhhT, bih, bhh, BM)

    gsum_n = _build_gather_sum(NP, n_rows, n_rows // C, C, NC, NS)
    nnm = gsum_n(msgs, n0, n1, n2)

    ne = _node_embedding(f_nuc.astype(F32), nnm, W1T, W2T, BM)
    ne_t = ne.reshape(B, L, H).transpose(1, 0, 2)

    T_b = 1
    for d in range(min(25, L), 0, -1):
        if L % d == 0:
            T_b = d
            break
    rep = _bilstm_maxpool(ne_t, WifT, WhfT, bf, WibT, WhbT, bb, T_b)
    return rep


# speedup vs baseline: 3.5904x; 1.0220x over previous
import functools

import jax
import jax.numpy as jnp
from jax import lax
from jax.experimental import pallas as pl
from jax.experimental.pallas import tpu as pltpu
from jax.experimental.pallas import tpu_sc as plsc

F32 = jnp.float32


def _sc_info():
    try:
        info = plsc.get_sparse_core_info()
        return info.num_cores, info.num_subcores
    except Exception:
        return 2, 16


def _build_gather_sum(P, n_rows, n_chunks, C, NC, NS):
    mesh = plsc.VectorSubcoreMesh(core_axis_name="c", subcore_axis_name="s")

    def body(msgs_hbm, i0_hbm, i1_hbm, i2_hbm, out_hbm,
             ia0, ia1, ia2, ib0, ib1, ib2, acc_v, gsem0, gsem1, wsem0, wsem1):
        wid = lax.axis_index("s") * NC + lax.axis_index("c")
        base0 = wid * n_rows
        idx_hbms = (i0_hbm, i1_hbm, i2_hbm)
        idx_vs = ((ia0, ia1, ia2), (ib0, ib1, ib2))
        gsems = (gsem0, gsem1)
        wsems = (wsem0, wsem1)

        def zero_stage_fire(koff, s):
            def rowz(r, c2):
                for l in range(8):
                    acc_v[s, r, pl.ds(l * 16, 16)] = jnp.zeros((16,), F32)
                return c2

            lax.fori_loop(0, C, rowz, 0)
            base = base0 + koff * C
            for j in range(3):
                pltpu.sync_copy(idx_hbms[j].at[pl.ds(base, C)], idx_vs[s][j])
            return [pltpu.async_copy(msgs_hbm.at[idx_vs[s][j]],
                                     acc_v.at[s], gsems[s], add=True)
                    for j in range(3)]

        def wb(koff, s):
            return pltpu.async_copy(acc_v.at[s],
                                    out_hbm.at[pl.ds(base0 + koff * C, C)],
                                    wsems[s])

        gh = [None] * n_chunks
        wbh = [None] * n_chunks
        gh[0] = zero_stage_fire(0, 0)
        for k in range(n_chunks):
            s = k % 2
            if k + 1 < n_chunks:
                if k >= 1:
                    wbh[k - 1].wait()
                gh[k + 1] = zero_stage_fire(k + 1, 1 - s)
            for h in gh[k]:
                h.wait()
            wbh[k] = wb(k, s)
        if n_chunks >= 2:
            wbh[n_chunks - 2].wait()
        wbh[n_chunks - 1].wait()

    return pl.kernel(
        body,
        out_type=jax.ShapeDtypeStruct((P, 128), F32),
        mesh=mesh,
        scratch_types=[
            pltpu.VMEM((C,), jnp.int32),
            pltpu.VMEM((C,), jnp.int32),
            pltpu.VMEM((C,), jnp.int32),
            pltpu.VMEM((C,), jnp.int32),
            pltpu.VMEM((C,), jnp.int32),
            pltpu.VMEM((C,), jnp.int32),
            pltpu.VMEM((2, C, 128), F32),
            pltpu.SemaphoreType.DMA,
            pltpu.SemaphoreType.DMA,
            pltpu.SemaphoreType.DMA,
            pltpu.SemaphoreType.DMA,
        ],
    )


def _build_gather_relu_sum(P, n_rows, n_chunks, C, NC, NS):
    relu = True
    mesh = plsc.VectorSubcoreMesh(core_axis_name="c", subcore_axis_name="s")

    def body(msgs_hbm, i0_hbm, i1_hbm, i2_hbm, out_hbm,
             ia0, ia1, ia2, ib0, ib1, ib2, rows_v, gsem0, gsem1, wsem0, wsem1):
        wid = lax.axis_index("s") * NC + lax.axis_index("c")
        base0 = wid * n_rows
        idx_hbms = (i0_hbm, i1_hbm, i2_hbm)
        idx_vs = ((ia0, ia1, ia2), (ib0, ib1, ib2))
        gsems = (gsem0, gsem1)
        wsems = (wsem0, wsem1)

        def stage_fire(koff, s):
            base = base0 + koff * C
            for j in range(3):
                pltpu.sync_copy(idx_hbms[j].at[pl.ds(base, C)], idx_vs[s][j])
            return [pltpu.async_copy(msgs_hbm.at[idx_vs[s][j]],
                                     rows_v.at[s, j], gsems[s])
                    for j in range(3)]

        def sum_wb(koff, s):
            def row(r, c2):
                for l in range(8):
                    sl = pl.ds(l * 16, 16)
                    if relu:
                        zero = jnp.zeros((16,), F32)
                        rows_v[s, 0, r, sl] = (
                            jnp.maximum(rows_v[s, 0, r, sl], zero)
                            + jnp.maximum(rows_v[s, 1, r, sl], zero)
                            + jnp.maximum(rows_v[s, 2, r, sl], zero))
                    else:
                        rows_v[s, 0, r, sl] = (rows_v[s, 0, r, sl]
                                               + rows_v[s, 1, r, sl]
                                               + rows_v[s, 2, r, sl])
                return c2

            lax.fori_loop(0, C, row, 0)
            return pltpu.async_copy(rows_v.at[s, 0],
                                    out_hbm.at[pl.ds(base0 + koff * C, C)],
                                    wsems[s])

        gh = [None] * n_chunks
        wbh = [None] * n_chunks
        gh[0] = stage_fire(0, 0)
        for k in range(n_chunks):
            s = k % 2
            if k + 1 < n_chunks:
                if k >= 1:
                    wbh[k - 1].wait()
                gh[k + 1] = stage_fire(k + 1, 1 - s)
            for h in gh[k]:
                h.wait()
            wbh[k] = sum_wb(k, s)
        if n_chunks >= 2:
            wbh[n_chunks - 2].wait()
        wbh[n_chunks - 1].wait()

    return pl.kernel(
        body,
        out_type=jax.ShapeDtypeStruct((P, 128), F32),
        mesh=mesh,
        scratch_types=[
            pltpu.VMEM((C,), jnp.int32),
            pltpu.VMEM((C,), jnp.int32),
            pltpu.VMEM((C,), jnp.int32),
            pltpu.VMEM((C,), jnp.int32),
            pltpu.VMEM((C,), jnp.int32),
            pltpu.VMEM((C,), jnp.int32),
            pltpu.VMEM((2, 3, C, 128), F32),
            pltpu.SemaphoreType.DMA,
            pltpu.SemaphoreType.DMA,
            pltpu.SemaphoreType.DMA,
            pltpu.SemaphoreType.DMA,
        ],
    )


def _local_potentials(f_bond, WlT, BM):
    E, K = f_bond.shape
    H = WlT.shape[1]

    def body(fb_ref, w_ref, lp_ref):
        lp_ref[...] = jnp.dot(fb_ref[...], w_ref[...],
                              preferred_element_type=F32)

    return pl.pallas_call(
        body,
        grid=(pl.cdiv(E, BM),),
        in_specs=[
            pl.BlockSpec((BM, K), lambda i: (i, 0)),
            pl.BlockSpec((K, H), lambda i: (0, 0)),
        ],
        out_specs=pl.BlockSpec((BM, H), lambda i: (i, 0)),
        out_shape=jax.ShapeDtypeStruct((E, H), F32),
    )(f_bond, WlT)


def _gru_update(sn, lp, msg, WmT, WihT, WhhT, bih, bhh, BM):
    E, H = lp.shape
    first = msg is None

    def body(sn_ref, lp_ref, *rest):
        if first:
            (wm_ref, wi_ref, wh_ref, bi_ref, bh_ref, out_ref) = rest
            h = jnp.maximum(lp_ref[...], 0.0)
        else:
            (msg_ref, wm_ref, wi_ref, wh_ref, bi_ref, bh_ref, out_ref) = rest
            h = msg_ref[...]
        nb = jnp.dot(sn_ref[...], wm_ref[...], preferred_element_type=F32)
        new = jnp.maximum(lp_ref[...] + nb, 0.0)
        gi = jnp.dot(new, wi_ref[...], preferred_element_type=F32) + bi_ref[...]
        gh = jnp.dot(h, wh_ref[...], preferred_element_type=F32) + bh_ref[...]
        r = jax.nn.sigmoid(gi[:, :H] + gh[:, :H])
        z = jax.nn.sigmoid(gi[:, H:2 * H] + gh[:, H:2 * H])
        n = jnp.tanh(gi[:, 2 * H:] + r * gh[:, 2 * H:])
        out_ref[...] = (1.0 - z) * n + z * h

        @pl.when(pl.program_id(0) == 0)
        def _():
            out_ref[0:1, :] = jnp.zeros((1, H), F32)

    blk = [pl.BlockSpec((BM, H), lambda i: (i, 0))]
    wspecs = [
        pl.BlockSpec((H, H), lambda i: (0, 0)),
        pl.BlockSpec((H, 3 * H), lambda i: (0, 0)),
        pl.BlockSpec((H, 3 * H), lambda i: (0, 0)),
        pl.BlockSpec((1, 3 * H), lambda i: (0, 0)),
        pl.BlockSpec((1, 3 * H), lambda i: (0, 0)),
    ]
    ins = (sn, lp) if first else (sn, lp, msg)
    return pl.pallas_call(
        body,
        grid=(pl.cdiv(E, BM),),
        in_specs=blk * len(ins) + wspecs,
        out_specs=pl.BlockSpec((BM, H), lambda i: (i, 0)),
        out_shape=jax.ShapeDtypeStruct((E, H), F32),
    )(*ins, WmT, WihT, WhhT, bih, bhh)


def _node_embedding(f_nuc_p, nnm, W1T, W2T, BM):
    NP, K = f_nuc_p.shape
    H = W2T.shape[1]

    def body(fn_ref, nm_ref, w1_ref, w2_ref, out_ref):
        acc = jnp.dot(fn_ref[...], w1_ref[...], preferred_element_type=F32)
        acc = acc + jnp.dot(nm_ref[...], w2_ref[...], preferred_element_type=F32)
        out_ref[...] = jnp.maximum(acc, 0.0)

    return pl.pallas_call(
        body,
        grid=(pl.cdiv(NP, BM),),
        in_specs=[
            pl.BlockSpec((BM, K), lambda i: (i, 0)),
            pl.BlockSpec((BM, H), lambda i: (i, 0)),
            pl.BlockSpec((K, H), lambda i: (0, 0)),
            pl.BlockSpec((H, H), lambda i: (0, 0)),
        ],
        out_specs=pl.BlockSpec((BM, H), lambda i: (i, 0)),
        out_shape=jax.ShapeDtypeStruct((NP, H), F32),
    )(f_nuc_p, nnm, W1T, W2T)


def _bilstm_maxpool(ne_t, WifT, WhfT, bf, WibT, WhbT, bb, T_b):
    Lq, Bq, H = ne_t.shape
    HH = WhfT.shape[0]
    G = Lq // T_b

    def body(nef_ref, neb_ref, wif_ref, whf_ref, bf_ref, wib_ref, whb_ref, bb_ref,
             out_ref, hf_s, cf_s, hb_s, cb_s, mf_s, mb_s):
        i = pl.program_id(0)

        @pl.when(i == 0)
        def _():
            z = jnp.zeros((Bq, HH), F32)
            hf_s[...] = z
            cf_s[...] = z
            hb_s[...] = z
            cb_s[...] = z
            m0 = jnp.full((Bq, HH), -jnp.inf, F32)
            mf_s[...] = m0
            mb_s[...] = m0

        def one_dir(x, h, c, wi_ref, wh_ref, b_ref):
            g = (jnp.dot(x, wi_ref[...], preferred_element_type=F32)
                 + jnp.dot(h, wh_ref[...], preferred_element_type=F32)
                 + b_ref[...])
            ig = jax.nn.sigmoid(g[:, :HH])
            fg = jax.nn.sigmoid(g[:, HH:2 * HH])
            gg = jnp.tanh(g[:, 2 * HH:3 * HH])
            og = jax.nn.sigmoid(g[:, 3 * HH:])
            c = fg * c + ig * gg
            h = og * jnp.tanh(c)
            return h, c

        def step(tt, carry):
            hf, cf, hb, cb, mf, mb = carry
            hf, cf = one_dir(nef_ref[tt], hf, cf, wif_ref, whf_ref, bf_ref)
            mf = jnp.maximum(mf, hf)
            hb, cb = one_dir(neb_ref[T_b - 1 - tt], hb, cb, wib_ref, whb_ref, bb_ref)
            mb = jnp.maximum(mb, hb)
            return hf, cf, hb, cb, mf, mb

        init = (hf_s[...], cf_s[...], hb_s[...], cb_s[...], mf_s[...], mb_s[...])
        hf, cf, hb, cb, mf, mb = lax.fori_loop(0, T_b, step, init)
        hf_s[...] = hf
        cf_s[...] = cf
        hb_s[...] = hb
        cb_s[...] = cb
        mf_s[...] = mf
        mb_s[...] = mb

        @pl.when(i == G - 1)
        def _():
            out_ref[...] = jnp.concatenate([mf, mb], axis=1)

    return pl.pallas_call(
        body,
        grid=(G,),
        in_specs=[
            pl.BlockSpec((T_b, Bq, H), lambda i: (i, 0, 0)),
            pl.BlockSpec((T_b, Bq, H), lambda i: (G - 1 - i, 0, 0)),
            pl.BlockSpec((H, 4 * HH), lambda i: (0, 0)),
            pl.BlockSpec((HH, 4 * HH), lambda i: (0, 0)),
            pl.BlockSpec((1, 4 * HH), lambda i: (0, 0)),
            pl.BlockSpec((H, 4 * HH), lambda i: (0, 0)),
            pl.BlockSpec((HH, 4 * HH), lambda i: (0, 0)),
            pl.BlockSpec((1, 4 * HH), lambda i: (0, 0)),
        ],
        out_specs=pl.BlockSpec((Bq, 2 * HH), lambda i: (0, 0)),
        out_shape=jax.ShapeDtypeStruct((Bq, 2 * HH), F32),
        scratch_shapes=[pltpu.VMEM((Bq, HH), F32)] * 6,
    )(ne_t, ne_t, WifT, WhfT, bf, WibT, WhbT, bb)


def _pad_rows(x, P):
    n = x.shape[0]
    if n == P:
        return x
    return jnp.concatenate(
        [x, jnp.zeros((P - n,) + x.shape[1:], x.dtype)], axis=0)


def kernel(f_nuc, f_bond, node_graph, message_graph, all_bonds, scope,
           W_local, W_msg, W_node_emb,
           gru_w_ih, gru_w_hh, gru_b_ih, gru_b_hh,
           lstm_w_ih_f, lstm_w_hh_f, lstm_b_ih_f, lstm_b_hh_f,
           lstm_w_ih_b, lstm_w_hh_b, lstm_b_ih_b, lstm_b_hh_b):
    E = f_bond.shape[0]
    N = f_nuc.shape[0]
    H = W_msg.shape[0]
    B = scope.shape[0]
    L = N // B
    HH = lstm_w_hh_f.shape[1]
    NC, NS = _sc_info()
    NW = NC * NS
    C = 128
    unit = NW * C

    EP = ((E + unit - 1) // unit) * unit
    NP = ((N + unit - 1) // unit) * unit
    e_rows = EP // NW
    n_rows = NP // NW

    def _pad_idx(x, P):
        n = x.shape[0]
        pad = (jnp.arange(P - n, dtype=jnp.int32) % E)[:, None]
        return jnp.concatenate(
            [x, jnp.broadcast_to(pad, (P - n, x.shape[1]))], axis=0)

    mg = _pad_idx(message_graph.astype(jnp.int32), EP)
    ng = _pad_idx(node_graph.astype(jnp.int32), NP)
    i0, i1, i2 = mg[:, 0], mg[:, 1], mg[:, 2]
    n0, n1, n2 = ng[:, 0], ng[:, 1], ng[:, 2]

    WlT = W_local.T.astype(F32)
    WmT = W_msg.T
    WihT = gru_w_ih.T
    WhhT = gru_w_hh.T
    bih = gru_b_ih.reshape(1, 3 * H)
    bhh = gru_b_hh.reshape(1, 3 * H)
    W1T = W_node_emb[:, :4].T
    W2T = W_node_emb[:, 4:].T
    WifT = lstm_w_ih_f.T
    WhfT = lstm_w_hh_f.T
    bf = (lstm_b_ih_f + lstm_b_hh_f).reshape(1, 4 * HH)
    WibT = lstm_w_ih_b.T
    WhbT = lstm_w_hh_b.T
    bb = (lstm_b_ih_b + lstm_b_hh_b).reshape(1, 4 * HH)

    BM = 2048
    lp = _local_potentials(f_bond.astype(F32), WlT, BM)

    gsum_e_relu = _build_gather_relu_sum(EP, e_rows, e_rows // C, C, NC, NS)
    gsum_e = _build_gather_sum(EP, e_rows, e_rows // C, C, NC, NS)
    sn = gsum_e_relu(lp, i0, i1, i2)
    msgs = _gru_update(sn, lp, None, WmT, WihT, WhhT, bih, bhh, BM)
    sn = gsum_e(msgs, i0, i1, i2)
    msgs = _gru_update(sn, lp, msgs, WmT, WihT, WhhT, bih, bhh, BM)

    gsum_n = _build_gather_sum(NP, n_rows, n_rows // C, C, NC, NS)
    nnm = gsum_n(msgs, n0, n1, n2)

    ne = _node_embedding(f_nuc.astype(F32), nnm, W1T, W2T, BM)
    ne_t = ne.reshape(B, L, H).transpose(1, 0, 2)

    T_b = 1
    for d in range(min(25, L), 0, -1):
        if L % d == 0:
            T_b = d
            break
    rep = _bilstm_maxpool(ne_t, WifT, WhfT, bf, WibT, WhbT, bb, T_b)
    return rep


# speedup vs baseline: 3.6029x; 1.0035x over previous
import functools

import jax
import jax.numpy as jnp
from jax import lax
from jax.experimental import pallas as pl
from jax.experimental.pallas import tpu as pltpu
from jax.experimental.pallas import tpu_sc as plsc

F32 = jnp.float32


def _sc_info():
    try:
        info = plsc.get_sparse_core_info()
        return info.num_cores, info.num_subcores
    except Exception:
        return 2, 16


def _build_gather_sum(P, n_rows, n_chunks, C, NC, NS):
    mesh = plsc.VectorSubcoreMesh(core_axis_name="c", subcore_axis_name="s")

    def body(msgs_hbm, i0_hbm, i1_hbm, i2_hbm, out_hbm,
             ia0, ia1, ia2, ib0, ib1, ib2, acc_v, gsem0, gsem1, wsem0, wsem1):
        wid = lax.axis_index("s") * NC + lax.axis_index("c")
        base0 = wid * n_rows
        idx_hbms = (i0_hbm, i1_hbm, i2_hbm)
        idx_vs = ((ia0, ia1, ia2), (ib0, ib1, ib2))
        gsems = (gsem0, gsem1)
        wsems = (wsem0, wsem1)

        def zero_stage_fire(koff, s):
            def rowz(r, c2):
                for l in range(8):
                    acc_v[s, r, pl.ds(l * 16, 16)] = jnp.zeros((16,), F32)
                return c2

            lax.fori_loop(0, C, rowz, 0)
            base = base0 + koff * C
            for j in range(3):
                pltpu.sync_copy(idx_hbms[j].at[pl.ds(base, C)], idx_vs[s][j])
            return [pltpu.async_copy(msgs_hbm.at[idx_vs[s][j]],
                                     acc_v.at[s], gsems[s], add=True)
                    for j in range(3)]

        def wb(koff, s):
            return pltpu.async_copy(acc_v.at[s],
                                    out_hbm.at[pl.ds(base0 + koff * C, C)],
                                    wsems[s])

        gh = [None] * n_chunks
        wbh = [None] * n_chunks
        gh[0] = zero_stage_fire(0, 0)
        for k in range(n_chunks):
            s = k % 2
            if k + 1 < n_chunks:
                if k >= 1:
                    wbh[k - 1].wait()
                gh[k + 1] = zero_stage_fire(k + 1, 1 - s)
            for h in gh[k]:
                h.wait()
            wbh[k] = wb(k, s)
        if n_chunks >= 2:
            wbh[n_chunks - 2].wait()
        wbh[n_chunks - 1].wait()

    return pl.kernel(
        body,
        out_type=jax.ShapeDtypeStruct((P, 128), F32),
        mesh=mesh,
        scratch_types=[
            pltpu.VMEM((C,), jnp.int32),
            pltpu.VMEM((C,), jnp.int32),
            pltpu.VMEM((C,), jnp.int32),
            pltpu.VMEM((C,), jnp.int32),
            pltpu.VMEM((C,), jnp.int32),
            pltpu.VMEM((C,), jnp.int32),
            pltpu.VMEM((2, C, 128), F32),
            pltpu.SemaphoreType.DMA,
            pltpu.SemaphoreType.DMA,
            pltpu.SemaphoreType.DMA,
            pltpu.SemaphoreType.DMA,
        ],
    )


def _build_gather_relu_sum(P, n_rows, n_chunks, C, NC, NS):
    relu = True
    mesh = plsc.VectorSubcoreMesh(core_axis_name="c", subcore_axis_name="s")

    def body(msgs_hbm, i0_hbm, i1_hbm, i2_hbm, out_hbm,
             ia0, ia1, ia2, ib0, ib1, ib2, rows_v, gsem0, gsem1, wsem0, wsem1):
        wid = lax.axis_index("s") * NC + lax.axis_index("c")
        base0 = wid * n_rows
        idx_hbms = (i0_hbm, i1_hbm, i2_hbm)
        idx_vs = ((ia0, ia1, ia2), (ib0, ib1, ib2))
        gsems = (gsem0, gsem1)
        wsems = (wsem0, wsem1)

        def stage_fire(koff, s):
            base = base0 + koff * C
            for j in range(3):
                pltpu.sync_copy(idx_hbms[j].at[pl.ds(base, C)], idx_vs[s][j])
            return [pltpu.async_copy(msgs_hbm.at[idx_vs[s][j]],
                                     rows_v.at[s, j], gsems[s])
                    for j in range(3)]

        def sum_wb(koff, s):
            def row(r, c2):
                for l in range(8):
                    sl = pl.ds(l * 16, 16)
                    if relu:
                        zero = jnp.zeros((16,), F32)
                        rows_v[s, 0, r, sl] = (
                            jnp.maximum(rows_v[s, 0, r, sl], zero)
                            + jnp.maximum(rows_v[s, 1, r, sl], zero)
                            + jnp.maximum(rows_v[s, 2, r, sl], zero))
                    else:
                        rows_v[s, 0, r, sl] = (rows_v[s, 0, r, sl]
                                               + rows_v[s, 1, r, sl]
                                               + rows_v[s, 2, r, sl])
                return c2

            lax.fori_loop(0, C, row, 0)
            return pltpu.async_copy(rows_v.at[s, 0],
                                    out_hbm.at[pl.ds(base0 + koff * C, C)],
                                    wsems[s])

        gh = [None] * n_chunks
        wbh = [None] * n_chunks
        gh[0] = stage_fire(0, 0)
        for k in range(n_chunks):
            s = k % 2
            if k + 1 < n_chunks:
                if k >= 1:
                    wbh[k - 1].wait()
                gh[k + 1] = stage_fire(k + 1, 1 - s)
            for h in gh[k]:
                h.wait()
            wbh[k] = sum_wb(k, s)
        if n_chunks >= 2:
            wbh[n_chunks - 2].wait()
        wbh[n_chunks - 1].wait()

    return pl.kernel(
        body,
        out_type=jax.ShapeDtypeStruct((P, 128), F32),
        mesh=mesh,
        scratch_types=[
            pltpu.VMEM((C,), jnp.int32),
            pltpu.VMEM((C,), jnp.int32),
            pltpu.VMEM((C,), jnp.int32),
            pltpu.VMEM((C,), jnp.int32),
            pltpu.VMEM((C,), jnp.int32),
            pltpu.VMEM((C,), jnp.int32),
            pltpu.VMEM((2, 3, C, 128), F32),
            pltpu.SemaphoreType.DMA,
            pltpu.SemaphoreType.DMA,
            pltpu.SemaphoreType.DMA,
            pltpu.SemaphoreType.DMA,
        ],
    )


def _local_potentials(f_bond, WlT, BM):
    E, K = f_bond.shape
    H = WlT.shape[1]

    def body(fb_ref, w_ref, lp_ref):
        lp_ref[...] = jnp.dot(fb_ref[...], w_ref[...],
                              preferred_element_type=F32)

    return pl.pallas_call(
        body,
        grid=(pl.cdiv(E, BM),),
        in_specs=[
            pl.BlockSpec((BM, K), lambda i: (i, 0)),
            pl.BlockSpec((K, H), lambda i: (0, 0)),
        ],
        out_specs=pl.BlockSpec((BM, H), lambda i: (i, 0)),
        out_shape=jax.ShapeDtypeStruct((E, H), F32),
    )(f_bond, WlT)


def _gru_update(sn, lp, msg, WmT, WihT, WhhT, bih, bhh, BM):
    E, H = lp.shape
    first = msg is None

    def body(sn_ref, lp_ref, *rest):
        if first:
            (wm_ref, wi_ref, wh_ref, bi_ref, bh_ref, out_ref) = rest
            h = jnp.maximum(lp_ref[...], 0.0)
        else:
            (msg_ref, wm_ref, wi_ref, wh_ref, bi_ref, bh_ref, out_ref) = rest
            h = msg_ref[...]
        nb = jnp.dot(sn_ref[...], wm_ref[...], preferred_element_type=F32)
        new = jnp.maximum(lp_ref[...] + nb, 0.0)
        gi = jnp.dot(new, wi_ref[...], preferred_element_type=F32) + bi_ref[...]
        gh = jnp.dot(h, wh_ref[...], preferred_element_type=F32) + bh_ref[...]
        r = jax.nn.sigmoid(gi[:, :H] + gh[:, :H])
        z = jax.nn.sigmoid(gi[:, H:2 * H] + gh[:, H:2 * H])
        n = jnp.tanh(gi[:, 2 * H:] + r * gh[:, 2 * H:])
        out_ref[...] = (1.0 - z) * n + z * h

        @pl.when(pl.program_id(0) == 0)
        def _():
            out_ref[0:1, :] = jnp.zeros((1, H), F32)

    blk = [pl.BlockSpec((BM, H), lambda i: (i, 0))]
    wspecs = [
        pl.BlockSpec((H, H), lambda i: (0, 0)),
        pl.BlockSpec((H, 3 * H), lambda i: (0, 0)),
        pl.BlockSpec((H, 3 * H), lambda i: (0, 0)),
        pl.BlockSpec((1, 3 * H), lambda i: (0, 0)),
        pl.BlockSpec((1, 3 * H), lambda i: (0, 0)),
    ]
    ins = (sn, lp) if first else (sn, lp, msg)
    return pl.pallas_call(
        body,
        grid=(pl.cdiv(E, BM),),
        in_specs=blk * len(ins) + wspecs,
        out_specs=pl.BlockSpec((BM, H), lambda i: (i, 0)),
        out_shape=jax.ShapeDtypeStruct((E, H), F32),
    )(*ins, WmT, WihT, WhhT, bih, bhh)


def _node_embedding(f_nuc_p, nnm, W1T, W2T, BM):
    NP, K = f_nuc_p.shape
    H = W2T.shape[1]

    def body(fn_ref, nm_ref, w1_ref, w2_ref, out_ref):
        acc = jnp.dot(fn_ref[...], w1_ref[...], preferred_element_type=F32)
        acc = acc + jnp.dot(nm_ref[...], w2_ref[...], preferred_element_type=F32)
        out_ref[...] = jnp.maximum(acc, 0.0)

    return pl.pallas_call(
        body,
        grid=(pl.cdiv(NP, BM),),
        in_specs=[
            pl.BlockSpec((BM, K), lambda i: (i, 0)),
            pl.BlockSpec((BM, H), lambda i: (i, 0)),
            pl.BlockSpec((K, H), lambda i: (0, 0)),
            pl.BlockSpec((H, H), lambda i: (0, 0)),
        ],
        out_specs=pl.BlockSpec((BM, H), lambda i: (i, 0)),
        out_shape=jax.ShapeDtypeStruct((NP, H), F32),
    )(f_nuc_p, nnm, W1T, W2T)


def _bilstm_maxpool(ne_t, WifT, WhfT, bf, WibT, WhbT, bb, T_b):
    Lq, Bq, H = ne_t.shape
    HH = WhfT.shape[0]
    G = Lq // T_b

    def body(nef_ref, neb_ref, wif_ref, whf_ref, bf_ref, wib_ref, whb_ref, bb_ref,
             out_ref, hf_s, cf_s, hb_s, cb_s, mf_s, mb_s, gxf_s, gxb_s):
        i = pl.program_id(0)

        for tt in range(T_b):
            gxf_s[tt] = jnp.dot(nef_ref[tt], wif_ref[...],
                                preferred_element_type=F32) + bf_ref[...]
            gxb_s[tt] = jnp.dot(neb_ref[T_b - 1 - tt], wib_ref[...],
                                preferred_element_type=F32) + bb_ref[...]

        @pl.when(i == 0)
        def _():
            z = jnp.zeros((Bq, HH), F32)
            hf_s[...] = z
            cf_s[...] = z
            hb_s[...] = z
            cb_s[...] = z
            m0 = jnp.full((Bq, HH), -jnp.inf, F32)
            mf_s[...] = m0
            mb_s[...] = m0

        def one_dir(gx, h, c, wh_ref):
            g = gx + jnp.dot(h, wh_ref[...], preferred_element_type=F32)
            ig = jax.nn.sigmoid(g[:, :HH])
            fg = jax.nn.sigmoid(g[:, HH:2 * HH])
            gg = jnp.tanh(g[:, 2 * HH:3 * HH])
            og = jax.nn.sigmoid(g[:, 3 * HH:])
            c = fg * c + ig * gg
            h = og * jnp.tanh(c)
            return h, c

        def step(tt, carry):
            hf, cf, hb, cb, mf, mb = carry
            hf, cf = one_dir(gxf_s[tt], hf, cf, whf_ref)
            mf = jnp.maximum(mf, hf)
            hb, cb = one_dir(gxb_s[tt], hb, cb, whb_ref)
            mb = jnp.maximum(mb, hb)
            return hf, cf, hb, cb, mf, mb

        init = (hf_s[...], cf_s[...], hb_s[...], cb_s[...], mf_s[...], mb_s[...])
        hf, cf, hb, cb, mf, mb = lax.fori_loop(0, T_b, step, init)
        hf_s[...] = hf
        cf_s[...] = cf
        hb_s[...] = hb
        cb_s[...] = cb
        mf_s[...] = mf
        mb_s[...] = mb

        @pl.when(i == G - 1)
        def _():
            out_ref[...] = jnp.concatenate([mf, mb], axis=1)

    return pl.pallas_call(
        body,
        grid=(G,),
        in_specs=[
            pl.BlockSpec((T_b, Bq, H), lambda i: (i, 0, 0)),
            pl.BlockSpec((T_b, Bq, H), lambda i: (G - 1 - i, 0, 0)),
            pl.BlockSpec((H, 4 * HH), lambda i: (0, 0)),
            pl.BlockSpec((HH, 4 * HH), lambda i: (0, 0)),
            pl.BlockSpec((1, 4 * HH), lambda i: (0, 0)),
            pl.BlockSpec((H, 4 * HH), lambda i: (0, 0)),
            pl.BlockSpec((HH, 4 * HH), lambda i: (0, 0)),
            pl.BlockSpec((1, 4 * HH), lambda i: (0, 0)),
        ],
        out_specs=pl.BlockSpec((Bq, 2 * HH), lambda i: (0, 0)),
        out_shape=jax.ShapeDtypeStruct((Bq, 2 * HH), F32),
        scratch_shapes=[pltpu.VMEM((Bq, HH), F32)] * 6
        + [pltpu.VMEM((T_b, Bq, 4 * HH), F32)] * 2,
    )(ne_t, ne_t, WifT, WhfT, bf, WibT, WhbT, bb)


def _pad_rows(x, P):
    n = x.shape[0]
    if n == P:
        return x
    return jnp.concatenate(
        [x, jnp.zeros((P - n,) + x.shape[1:], x.dtype)], axis=0)


def kernel(f_nuc, f_bond, node_graph, message_graph, all_bonds, scope,
           W_local, W_msg, W_node_emb,
           gru_w_ih, gru_w_hh, gru_b_ih, gru_b_hh,
           lstm_w_ih_f, lstm_w_hh_f, lstm_b_ih_f, lstm_b_hh_f,
           lstm_w_ih_b, lstm_w_hh_b, lstm_b_ih_b, lstm_b_hh_b):
    E = f_bond.shape[0]
    N = f_nuc.shape[0]
    H = W_msg.shape[0]
    B = scope.shape[0]
    L = N // B
    HH = lstm_w_hh_f.shape[1]
    NC, NS = _sc_info()
    NW = NC * NS
    C = 128
    unit = NW * C

    EP = ((E + unit - 1) // unit) * unit
    NP = ((N + unit - 1) // unit) * unit
    e_rows = EP // NW
    n_rows = NP // NW

    def _pad_idx(x, P):
        n = x.shape[0]
        pad = (jnp.arange(P - n, dtype=jnp.int32) % E)[:, None]
        return jnp.concatenate(
            [x, jnp.broadcast_to(pad, (P - n, x.shape[1]))], axis=0)

    mg = _pad_idx(message_graph.astype(jnp.int32), EP)
    ng = _pad_idx(node_graph.astype(jnp.int32), NP)
    i0, i1, i2 = mg[:, 0], mg[:, 1], mg[:, 2]
    n0, n1, n2 = ng[:, 0], ng[:, 1], ng[:, 2]

    WlT = W_local.T.astype(F32)
    WmT = W_msg.T
    WihT = gru_w_ih.T
    WhhT = gru_w_hh.T
    bih = gru_b_ih.reshape(1, 3 * H)
    bhh = gru_b_hh.reshape(1, 3 * H)
    W1T = W_node_emb[:, :4].T
    W2T = W_node_emb[:, 4:].T
    WifT = lstm_w_ih_f.T
    WhfT = lstm_w_hh_f.T
    bf = (lstm_b_ih_f + lstm_b_hh_f).reshape(1, 4 * HH)
    WibT = lstm_w_ih_b.T
    WhbT = lstm_w_hh_b.T
    bb = (lstm_b_ih_b + lstm_b_hh_b).reshape(1, 4 * HH)

    BM = 2048
    lp = _local_potentials(f_bond.astype(F32), WlT, BM)

    gsum_e_relu = _build_gather_relu_sum(EP, e_rows, e_rows // C, C, NC, NS)
    gsum_e = _build_gather_sum(EP, e_rows, e_rows // C, C, NC, NS)
    sn = gsum_e_relu(lp, i0, i1, i2)
    msgs = _gru_update(sn, lp, None, WmT, WihT, WhhT, bih, bhh, BM)
    sn = gsum_e(msgs, i0, i1, i2)
    msgs = _gru_update(sn, lp, msgs, WmT, WihT, WhhT, bih, bhh, BM)

    gsum_n = _build_gather_sum(NP, n_rows, n_rows // C, C, NC, NS)
    nnm = gsum_n(msgs, n0, n1, n2)

    ne = _node_embedding(f_nuc.astype(F32), nnm, W1T, W2T, BM)
    ne_t = ne.reshape(B, L, H).transpose(1, 0, 2)

    T_b = 1
    for d in range(min(25, L), 0, -1):
        if L % d == 0:
            T_b = d
            break
    rep = _bilstm_maxpool(ne_t, WifT, WhfT, bf, WibT, WhbT, bb, T_b)
    return rep
